# Initial kernel scaffold; baseline (speedup 1.0000x reference)
import jax
import jax.numpy as jnp
from jax.experimental import pallas as pl


def kernel(x, edge_index, params):
    raise NotImplementedError("write your pallas kernel here")


# speedup vs baseline: 11.3151x; 11.3151x over previous
import functools

import jax
import jax.numpy as jnp
from jax import lax
from jax.experimental import pallas as pl
from jax.experimental.pallas import tpu as pltpu
from jax.experimental.pallas import tpu_sc as plsc

N = 10000
E = 320000
D = 128
H = 128
C = 40
NUM_LAYERS = 3

_CHUNK = 128
_NCHUNK = E // _CHUNK
_NW = 32
_RPT = 624
_ZCH = 208
_TAIL = 16
_TAIL_BASE = 16 * _RPT

_BLK = 1000
_GRID = N // _BLK


def _edge_range(w):
    lo = (w * _NCHUNK) // _NW
    hi = ((w + 1) * _NCHUNK) // _NW
    return lo, hi


@functools.cache
def _get_sc_degree():
    return functools.partial(
        pl.kernel,
        out_type=jax.ShapeDtypeStruct((2, N, H), jnp.float32),
        mesh=plsc.VectorSubcoreMesh(core_axis_name="c", subcore_axis_name="s"),
        scratch_types=[
            pltpu.VMEM((_CHUNK,), jnp.int32),
            pltpu.VMEM((_CHUNK, H), jnp.float32),
            pltpu.VMEM((_ZCH, H), jnp.float32),
            pltpu.VMEM_SHARED((N, H), jnp.float32),
        ],
    )(_sc_degree_body)


def _sc_degree_body(dst_hbm, ones_hbm, zeros_hbm, out_hbm, dst_v, ones_v, zero_v, acc):
    c = lax.axis_index("c")
    s = lax.axis_index("s")
    pltpu.sync_copy(zeros_hbm, zero_v)
    pltpu.sync_copy(ones_hbm, ones_v)
    for j in range(_RPT // _ZCH):
        pltpu.sync_copy(zero_v, acc.at[pl.ds(s * _RPT + j * _ZCH, _ZCH)])

    @pl.when(s == 15)
    def _zero_tail():
        pltpu.sync_copy(zero_v.at[pl.ds(0, _TAIL)], acc.at[pl.ds(_TAIL_BASE, _TAIL)])

    plsc.subcore_barrier()
    w = s * 2 + c
    lo, hi = _edge_range(w)

    def body(i, carry):
        pltpu.sync_copy(dst_hbm.at[pl.ds(i * _CHUNK, _CHUNK)], dst_v)
        pltpu.sync_copy(ones_v, acc.at[dst_v], add=True)
        return carry

    lax.fori_loop(lo, hi, body, 0)
    plsc.subcore_barrier()
    pltpu.sync_copy(acc.at[pl.ds(s * _RPT, _RPT)],
                    out_hbm.at[c, pl.ds(s * _RPT, _RPT)])

    @pl.when(s == 15)
    def _copy_tail():
        pltpu.sync_copy(acc.at[pl.ds(_TAIL_BASE, _TAIL)],
                        out_hbm.at[c, pl.ds(_TAIL_BASE, _TAIL)])


@functools.cache
def _get_sc_aggregate():
    return functools.partial(
        pl.kernel,
        out_type=jax.ShapeDtypeStruct((2, N, H), jnp.float32),
        mesh=plsc.VectorSubcoreMesh(core_axis_name="c", subcore_axis_name="s"),
        scratch_types=[
            pltpu.VMEM((_CHUNK,), jnp.int32),
            pltpu.VMEM((_CHUNK,), jnp.int32),
            pltpu.VMEM((_CHUNK, H), jnp.float32),
            pltpu.VMEM((_ZCH, H), jnp.float32),
            pltpu.VMEM_SHARED((N, H), jnp.float32),
            pltpu.SemaphoreType.DMA,
        ],
    )(_sc_aggregate_body)


def _sc_aggregate_body(hwp_hbm, src_hbm, dst_hbm, zeros_hbm, out_hbm,
                       src_v, dst_v, rows_v, zero_v, acc, sem):
    c = lax.axis_index("c")
    s = lax.axis_index("s")
    pltpu.sync_copy(zeros_hbm, zero_v)
    for j in range(_RPT // _ZCH):
        pltpu.sync_copy(zero_v, acc.at[pl.ds(s * _RPT + j * _ZCH, _ZCH)])

    @pl.when(s == 15)
    def _zero_tail():
        pltpu.sync_copy(zero_v.at[pl.ds(0, _TAIL)], acc.at[pl.ds(_TAIL_BASE, _TAIL)])

    plsc.subcore_barrier()
    w = s * 2 + c
    lo, hi = _edge_range(w)

    def body(i, carry):
        base = i * _CHUNK
        pltpu.sync_copy(src_hbm.at[pl.ds(base, _CHUNK)], src_v)
        pltpu.sync_copy(dst_hbm.at[pl.ds(base, _CHUNK)], dst_v)
        pltpu.async_copy(hwp_hbm.at[src_v], rows_v, sem).wait()
        pltpu.sync_copy(rows_v, acc.at[dst_v], add=True)
        return carry

    lax.fori_loop(lo, hi, body, 0)
    plsc.subcore_barrier()
    pltpu.sync_copy(acc.at[pl.ds(s * _RPT, _RPT)],
                    out_hbm.at[c, pl.ds(s * _RPT, _RPT)])

    @pl.when(s == 15)
    def _copy_tail():
        pltpu.sync_copy(acc.at[pl.ds(_TAIL_BASE, _TAIL)],
                        out_hbm.at[c, pl.ds(_TAIL_BASE, _TAIL)])


def _dinv_block(d0, d1):
    deg = 1.0 + d0[:, 0:1] + d1[:, 0:1]
    return lax.rsqrt(deg)


def _a0_body(la_ref, lb_ref, x1_ref, x2_ref, w1_ref, b1_ref, w2_ref,
             d0_ref, d1_ref, out_ref):
    alpha = jnp.exp(la_ref[0, 0])
    beta = jnp.exp(lb_ref[0, 0])
    h = alpha * x1_ref[...] + beta * x2_ref[...]
    m = jnp.maximum(jnp.dot(h, w1_ref[...], preferred_element_type=jnp.float32)
                    + b1_ref[...], 0.0)
    hw = jnp.dot(m, w2_ref[...], preferred_element_type=jnp.float32)
    out_ref[...] = hw * _dinv_block(d0_ref[...], d1_ref[...])


def _a_body(h_ref, w1_ref, b1_ref, w2_ref, d0_ref, d1_ref, out_ref):
    m = jnp.maximum(jnp.dot(h_ref[...], w1_ref[...],
                            preferred_element_type=jnp.float32) + b1_ref[...], 0.0)
    hw = jnp.dot(m, w2_ref[...], preferred_element_type=jnp.float32)
    out_ref[...] = hw * _dinv_block(d0_ref[...], d1_ref[...])


def _b1_body(a0_ref, a1_ref, hwp_ref, d0_ref, d1_ref, cb_ref,
             t_ref, s_ref, ss_ref):
    dinv = _dinv_block(d0_ref[...], d1_ref[...])
    t = dinv * (a0_ref[...] + a1_ref[...] + hwp_ref[...]) + cb_ref[...]
    t_ref[...] = t
    s_ref[...] = jnp.sum(t, axis=0, keepdims=True)[None]
    ss_ref[...] = jnp.sum(t * t, axis=0, keepdims=True)[None]


def _b2_body(t_ref, s_ref, ss_ref, g_ref, b_ref, out_ref):
    ssum = jnp.sum(s_ref[...], axis=0)
    sqsum = jnp.sum(ss_ref[...], axis=0)
    mean = ssum * (1.0 / N)
    var = sqsum * (1.0 / N) - mean * mean
    y = (t_ref[...] - mean) / jnp.sqrt(var + 1e-5) * g_ref[...] + b_ref[...]
    out_ref[...] = jnp.maximum(y, 0.0)


def _blast_body(a0_ref, a1_ref, hwp_ref, d0_ref, d1_ref, cb_ref,
                lw_ref, lb_ref, out_ref):
    dinv = _dinv_block(d0_ref[...], d1_ref[...])
    t = dinv * (a0_ref[...] + a1_ref[...] + hwp_ref[...]) + cb_ref[...]
    z = jnp.dot(t, lw_ref[...], preferred_element_type=jnp.float32) + lb_ref[...]
    z = z - jnp.max(z, axis=-1, keepdims=True)
    out_ref[...] = z - jnp.log(jnp.sum(jnp.exp(z), axis=-1, keepdims=True))


def _row_spec(width):
    return pl.BlockSpec((_BLK, width), lambda i: (i, 0))


def _full_spec(shape):
    return pl.BlockSpec(shape, lambda i: tuple(0 for _ in shape))


_SMEM_SPEC = pl.BlockSpec(memory_space=pltpu.SMEM)


def _tc_a0(la, lb, x1, x2, w1, b1, w2, d0, d1):
    return pl.pallas_call(
        _a0_body,
        grid=(_GRID,),
        in_specs=[_SMEM_SPEC, _SMEM_SPEC, _row_spec(D), _row_spec(D),
                  _full_spec((D, H)), _full_spec((1, H)), _full_spec((H, H)),
                  _row_spec(H), _row_spec(H)],
        out_specs=_row_spec(H),
        out_shape=jax.ShapeDtypeStruct((N, H), jnp.float32),
    )(la, lb, x1, x2, w1, b1, w2, d0, d1)


def _tc_a(h, w1, b1, w2, d0, d1):
    return pl.pallas_call(
        _a_body,
        grid=(_GRID,),
        in_specs=[_row_spec(H), _full_spec((H, H)), _full_spec((1, H)),
                  _full_spec((H, H)), _row_spec(H), _row_spec(H)],
        out_specs=_row_spec(H),
        out_shape=jax.ShapeDtypeStruct((N, H), jnp.float32),
    )(h, w1, b1, w2, d0, d1)


def _tc_b1(a0, a1, hwp, d0, d1, cb):
    return pl.pallas_call(
        _b1_body,
        grid=(_GRID,),
        in_specs=[_row_spec(H), _row_spec(H), _row_spec(H),
                  _row_spec(H), _row_spec(H), _full_spec((1, H))],
        out_specs=[_row_spec(H),
                   pl.BlockSpec((1, 1, H), lambda i: (i, 0, 0)),
                   pl.BlockSpec((1, 1, H), lambda i: (i, 0, 0))],
        out_shape=[jax.ShapeDtypeStruct((N, H), jnp.float32),
                   jax.ShapeDtypeStruct((_GRID, 1, H), jnp.float32),
                   jax.ShapeDtypeStruct((_GRID, 1, H), jnp.float32)],
    )(a0, a1, hwp, d0, d1, cb)


def _tc_b2(t, s, ss, g, b):
    return pl.pallas_call(
        _b2_body,
        grid=(_GRID,),
        in_specs=[_row_spec(H), _full_spec((_GRID, 1, H)), _full_spec((_GRID, 1, H)),
                  _full_spec((1, H)), _full_spec((1, H))],
        out_specs=_row_spec(H),
        out_shape=jax.ShapeDtypeStruct((N, H), jnp.float32),
    )(t, s, ss, g, b)


def _tc_blast(a0, a1, hwp, d0, d1, cb, lw, lb):
    return pl.pallas_call(
        _blast_body,
        grid=(_GRID,),
        in_specs=[_row_spec(H), _row_spec(H), _row_spec(H),
                  _row_spec(H), _row_spec(H), _full_spec((1, H)),
                  _full_spec((H, C)), _full_spec((1, C))],
        out_specs=_row_spec(C),
        out_shape=jax.ShapeDtypeStruct((N, C), jnp.float32),
    )(a0, a1, hwp, d0, d1, cb, lw, lb)


def kernel(x, edge_index, params):
    src = edge_index[0]
    dst = edge_index[1]
    x1 = x[:, :D]
    x2 = x[:, D:]

    ones128 = jnp.ones((_CHUNK, H), jnp.float32)
    zeros128 = jnp.zeros((_ZCH, H), jnp.float32)

    degp = _get_sc_degree()(dst, ones128, zeros128)
    d0 = degp[0]
    d1 = degp[1]

    la = params["log_alpha"].reshape(1, 1)
    lb = params["log_beta"].reshape(1, 1)

    h = None
    out = None
    for i in range(NUM_LAYERS):
        w1 = params["mlp_w"][i]
        b1 = params["mlp_b"][i].reshape(1, H)
        w2 = params["conv_w"][i]
        cb = params["conv_b"][i].reshape(1, H)
        if i == 0:
            hwp = _tc_a0(la, lb, x1, x2, w1, b1, w2, d0, d1)
        else:
            hwp = _tc_a(h, w1, b1, w2, d0, d1)
        agg = _get_sc_aggregate()(hwp, src, dst, zeros128)
        if i < NUM_LAYERS - 1:
            t, s, ss = _tc_b1(agg[0], agg[1], hwp, d0, d1, cb)
            h = _tc_b2(t, s, ss,
                       params["bn_gamma"][i].reshape(1, H),
                       params["bn_beta"][i].reshape(1, H))
        else:
            out = _tc_blast(agg[0], agg[1], hwp, d0, d1, cb,
                            params["lin_w"], params["lin_b"].reshape(1, C))
    return out


# speedup vs baseline: 14.4156x; 1.2740x over previous
import functools

import jax
import jax.numpy as jnp
from jax import lax
from jax.experimental import pallas as pl
from jax.experimental.pallas import tpu as pltpu
from jax.experimental.pallas import tpu_sc as plsc

N = 10000
E = 320000
D = 128
H = 128
C = 40
NUM_LAYERS = 3

_CHUNK = 128
_NCHUNK = E // _CHUNK
_NW = 32
_RPT = 624
_ZCH = 104
_TAIL = 16
_TAIL_BASE = 16 * _RPT

_BLK = 1000
_GRID = N // _BLK


def _edge_range(w):
    lo = (w * _NCHUNK) // _NW
    hi = ((w + 1) * _NCHUNK) // _NW
    return lo, hi


@functools.cache
def _get_sc_degree():
    return functools.partial(
        pl.kernel,
        out_type=jax.ShapeDtypeStruct((2, N, H), jnp.float32),
        mesh=plsc.VectorSubcoreMesh(core_axis_name="c", subcore_axis_name="s"),
        scratch_types=[
            pltpu.VMEM((_CHUNK,), jnp.int32),
            pltpu.VMEM((_CHUNK, H), jnp.float32),
            pltpu.VMEM((_ZCH, H), jnp.float32),
            pltpu.VMEM_SHARED((N, H), jnp.float32),
        ],
    )(_sc_degree_body)


def _sc_degree_body(dst_hbm, ones_hbm, zeros_hbm, out_hbm, dst_v, ones_v, zero_v, acc):
    c = lax.axis_index("c")
    s = lax.axis_index("s")
    pltpu.sync_copy(zeros_hbm, zero_v)
    pltpu.sync_copy(ones_hbm, ones_v)
    for j in range(_RPT // _ZCH):
        pltpu.sync_copy(zero_v, acc.at[pl.ds(s * _RPT + j * _ZCH, _ZCH)])

    @pl.when(s == 15)
    def _zero_tail():
        pltpu.sync_copy(zero_v.at[pl.ds(0, _TAIL)], acc.at[pl.ds(_TAIL_BASE, _TAIL)])

    plsc.subcore_barrier()
    w = s * 2 + c
    lo, hi = _edge_range(w)

    def body(i, carry):
        pltpu.sync_copy(dst_hbm.at[pl.ds(i * _CHUNK, _CHUNK)], dst_v)
        pltpu.sync_copy(ones_v, acc.at[dst_v], add=True)
        return carry

    lax.fori_loop(lo, hi, body, 0)
    plsc.subcore_barrier()
    pltpu.sync_copy(acc.at[pl.ds(s * _RPT, _RPT)],
                    out_hbm.at[c, pl.ds(s * _RPT, _RPT)])

    @pl.when(s == 15)
    def _copy_tail():
        pltpu.sync_copy(acc.at[pl.ds(_TAIL_BASE, _TAIL)],
                        out_hbm.at[c, pl.ds(_TAIL_BASE, _TAIL)])


@functools.cache
def _get_sc_aggregate():
    return functools.partial(
        pl.kernel,
        out_type=jax.ShapeDtypeStruct((2, N, H), jnp.float32),
        mesh=plsc.VectorSubcoreMesh(core_axis_name="c", subcore_axis_name="s"),
        scratch_types=[
            pltpu.VMEM((_CHUNK,), jnp.int32),
            pltpu.VMEM((_CHUNK,), jnp.int32),
            pltpu.VMEM((_CHUNK,), jnp.int32),
            pltpu.VMEM((_CHUNK,), jnp.int32),
            pltpu.VMEM((_CHUNK, H), jnp.float32),
            pltpu.VMEM((_CHUNK, H), jnp.float32),
            pltpu.VMEM((_ZCH, H), jnp.float32),
            pltpu.VMEM_SHARED((N, H), jnp.float32),
            pltpu.SemaphoreType.DMA,
            pltpu.SemaphoreType.DMA,
            pltpu.SemaphoreType.DMA,
            pltpu.SemaphoreType.DMA,
        ],
    )(_sc_aggregate_body)


def _sc_aggregate_body(hwp_hbm, src_hbm, dst_hbm, zeros_hbm, out_hbm,
                       src0, src1, dst0, dst1, rows0, rows1, zero_v, acc,
                       sem_i0, sem_i1, sem_g0, sem_g1):
    c = lax.axis_index("c")
    s = lax.axis_index("s")
    src_v = (src0, src1)
    dst_v = (dst0, dst1)
    rows_v = (rows0, rows1)
    sem_i = (sem_i0, sem_i1)
    sem_g = (sem_g0, sem_g1)

    pltpu.sync_copy(zeros_hbm, zero_v)
    for j in range(_RPT // _ZCH):
        pltpu.sync_copy(zero_v, acc.at[pl.ds(s * _RPT + j * _ZCH, _ZCH)])

    @pl.when(s == 15)
    def _zero_tail():
        pltpu.sync_copy(zero_v.at[pl.ds(0, _TAIL)], acc.at[pl.ds(_TAIL_BASE, _TAIL)])

    plsc.subcore_barrier()
    w = s * 2 + c
    lo, hi = _edge_range(w)

    def load_idx(i, b):
        pltpu.async_copy(src_hbm.at[pl.ds(i * _CHUNK, _CHUNK)], src_v[b], sem_i[b])
        pltpu.async_copy(dst_hbm.at[pl.ds(i * _CHUNK, _CHUNK)], dst_v[b], sem_i[b])

    def wait_idx(b):
        pltpu.make_async_copy(src_hbm.at[pl.ds(0, _CHUNK)], src_v[b], sem_i[b]).wait()
        pltpu.make_async_copy(dst_hbm.at[pl.ds(0, _CHUNK)], dst_v[b], sem_i[b]).wait()

    def start_gather(b):
        pltpu.async_copy(hwp_hbm.at[src_v[b]], rows_v[b], sem_g[b])

    def wait_gather(b):
        pltpu.make_async_copy(hwp_hbm.at[pl.ds(0, _CHUNK)], rows_v[b], sem_g[b]).wait()

    load_idx(lo, 0)
    load_idx(lo + 1, 1)
    wait_idx(0)
    start_gather(0)

    def step(i, b, nb):
        wait_gather(b)
        pltpu.sync_copy(rows_v[b], acc.at[dst_v[b]], add=True)

        @pl.when(i + 2 < hi)
        def _prefetch_idx():
            load_idx(i + 2, b)

        @pl.when(i + 1 < hi)
        def _next_gather():
            wait_idx(nb)
            start_gather(nb)

    def pair(k, carry):
        i0 = lo + 2 * k
        step(i0, 0, 1)
        step(i0 + 1, 1, 0)
        return carry

    npairs = (hi - lo) // 2
    lax.fori_loop(0, npairs, pair, 0)

    @pl.when((hi - lo) % 2 == 1)
    def _tail_step():
        step(hi - 1, 0, 1)

    plsc.subcore_barrier()
    pltpu.sync_copy(acc.at[pl.ds(s * _RPT, _RPT)],
                    out_hbm.at[c, pl.ds(s * _RPT, _RPT)])

    @pl.when(s == 15)
    def _copy_tail():
        pltpu.sync_copy(acc.at[pl.ds(_TAIL_BASE, _TAIL)],
                        out_hbm.at[c, pl.ds(_TAIL_BASE, _TAIL)])


def _dinv_block(d0, d1):
    deg = 1.0 + d0[:, 0:1] + d1[:, 0:1]
    return lax.rsqrt(deg)


def _a0_body(la_ref, lb_ref, x1_ref, x2_ref, w1_ref, b1_ref, w2_ref,
             d0_ref, d1_ref, out_ref):
    alpha = jnp.exp(la_ref[0, 0])
    beta = jnp.exp(lb_ref[0, 0])
    h = alpha * x1_ref[...] + beta * x2_ref[...]
    m = jnp.maximum(jnp.dot(h, w1_ref[...], preferred_element_type=jnp.float32)
                    + b1_ref[...], 0.0)
    hw = jnp.dot(m, w2_ref[...], preferred_element_type=jnp.float32)
    out_ref[...] = hw * _dinv_block(d0_ref[...], d1_ref[...])


def _a_body(h_ref, w1_ref, b1_ref, w2_ref, d0_ref, d1_ref, out_ref):
    m = jnp.maximum(jnp.dot(h_ref[...], w1_ref[...],
                            preferred_element_type=jnp.float32) + b1_ref[...], 0.0)
    hw = jnp.dot(m, w2_ref[...], preferred_element_type=jnp.float32)
    out_ref[...] = hw * _dinv_block(d0_ref[...], d1_ref[...])


def _b1_body(a0_ref, a1_ref, hwp_ref, d0_ref, d1_ref, cb_ref,
             t_ref, s_ref, ss_ref):
    dinv = _dinv_block(d0_ref[...], d1_ref[...])
    t = dinv * (a0_ref[...] + a1_ref[...] + hwp_ref[...]) + cb_ref[...]
    t_ref[...] = t
    s_ref[...] = jnp.sum(t, axis=0, keepdims=True)[None]
    ss_ref[...] = jnp.sum(t * t, axis=0, keepdims=True)[None]


def _b2_body(t_ref, s_ref, ss_ref, g_ref, b_ref, out_ref):
    ssum = jnp.sum(s_ref[...], axis=0)
    sqsum = jnp.sum(ss_ref[...], axis=0)
    mean = ssum * (1.0 / N)
    var = sqsum * (1.0 / N) - mean * mean
    y = (t_ref[...] - mean) / jnp.sqrt(var + 1e-5) * g_ref[...] + b_ref[...]
    out_ref[...] = jnp.maximum(y, 0.0)


def _blast_body(a0_ref, a1_ref, hwp_ref, d0_ref, d1_ref, cb_ref,
                lw_ref, lb_ref, out_ref):
    dinv = _dinv_block(d0_ref[...], d1_ref[...])
    t = dinv * (a0_ref[...] + a1_ref[...] + hwp_ref[...]) + cb_ref[...]
    z = jnp.dot(t, lw_ref[...], preferred_element_type=jnp.float32) + lb_ref[...]
    z = z - jnp.max(z, axis=-1, keepdims=True)
    out_ref[...] = z - jnp.log(jnp.sum(jnp.exp(z), axis=-1, keepdims=True))


def _row_spec(width):
    return pl.BlockSpec((_BLK, width), lambda i: (i, 0))


def _full_spec(shape):
    return pl.BlockSpec(shape, lambda i: tuple(0 for _ in shape))


_SMEM_SPEC = pl.BlockSpec(memory_space=pltpu.SMEM)


def _tc_a0(la, lb, x1, x2, w1, b1, w2, d0, d1):
    return pl.pallas_call(
        _a0_body,
        grid=(_GRID,),
        in_specs=[_SMEM_SPEC, _SMEM_SPEC, _row_spec(D), _row_spec(D),
                  _full_spec((D, H)), _full_spec((1, H)), _full_spec((H, H)),
                  _row_spec(H), _row_spec(H)],
        out_specs=_row_spec(H),
        out_shape=jax.ShapeDtypeStruct((N, H), jnp.float32),
    )(la, lb, x1, x2, w1, b1, w2, d0, d1)


def _tc_a(h, w1, b1, w2, d0, d1):
    return pl.pallas_call(
        _a_body,
        grid=(_GRID,),
        in_specs=[_row_spec(H), _full_spec((H, H)), _full_spec((1, H)),
                  _full_spec((H, H)), _row_spec(H), _row_spec(H)],
        out_specs=_row_spec(H),
        out_shape=jax.ShapeDtypeStruct((N, H), jnp.float32),
    )(h, w1, b1, w2, d0, d1)


def _tc_b1(a0, a1, hwp, d0, d1, cb):
    return pl.pallas_call(
        _b1_body,
        grid=(_GRID,),
        in_specs=[_row_spec(H), _row_spec(H), _row_spec(H),
                  _row_spec(H), _row_spec(H), _full_spec((1, H))],
        out_specs=[_row_spec(H),
                   pl.BlockSpec((1, 1, H), lambda i: (i, 0, 0)),
                   pl.BlockSpec((1, 1, H), lambda i: (i, 0, 0))],
        out_shape=[jax.ShapeDtypeStruct((N, H), jnp.float32),
                   jax.ShapeDtypeStruct((_GRID, 1, H), jnp.float32),
                   jax.ShapeDtypeStruct((_GRID, 1, H), jnp.float32)],
    )(a0, a1, hwp, d0, d1, cb)


def _tc_b2(t, s, ss, g, b):
    return pl.pallas_call(
        _b2_body,
        grid=(_GRID,),
        in_specs=[_row_spec(H), _full_spec((_GRID, 1, H)), _full_spec((_GRID, 1, H)),
                  _full_spec((1, H)), _full_spec((1, H))],
        out_specs=_row_spec(H),
        out_shape=jax.ShapeDtypeStruct((N, H), jnp.float32),
    )(t, s, ss, g, b)


def _tc_blast(a0, a1, hwp, d0, d1, cb, lw, lb):
    return pl.pallas_call(
        _blast_body,
        grid=(_GRID,),
        in_specs=[_row_spec(H), _row_spec(H), _row_spec(H),
                  _row_spec(H), _row_spec(H), _full_spec((1, H)),
                  _full_spec((H, C)), _full_spec((1, C))],
        out_specs=_row_spec(C),
        out_shape=jax.ShapeDtypeStruct((N, C), jnp.float32),
    )(a0, a1, hwp, d0, d1, cb, lw, lb)


def kernel(x, edge_index, params):
    src = edge_index[0]
    dst = edge_index[1]
    x1 = x[:, :D]
    x2 = x[:, D:]

    ones128 = jnp.ones((_CHUNK, H), jnp.float32)
    zeros128 = jnp.zeros((_ZCH, H), jnp.float32)

    degp = _get_sc_degree()(dst, ones128, zeros128)
    d0 = degp[0]
    d1 = degp[1]

    la = params["log_alpha"].reshape(1, 1)
    lb = params["log_beta"].reshape(1, 1)

    h = None
    out = None
    for i in range(NUM_LAYERS):
        w1 = params["mlp_w"][i]
        b1 = params["mlp_b"][i].reshape(1, H)
        w2 = params["conv_w"][i]
        cb = params["conv_b"][i].reshape(1, H)
        if i == 0:
            hwp = _tc_a0(la, lb, x1, x2, w1, b1, w2, d0, d1)
        else:
            hwp = _tc_a(h, w1, b1, w2, d0, d1)
        agg = _get_sc_aggregate()(hwp, src, dst, zeros128)
        if i < NUM_LAYERS - 1:
            t, s, ss = _tc_b1(agg[0], agg[1], hwp, d0, d1, cb)
            h = _tc_b2(t, s, ss,
                       params["bn_gamma"][i].reshape(1, H),
                       params["bn_beta"][i].reshape(1, H))
        else:
            out = _tc_blast(agg[0], agg[1], hwp, d0, d1, cb,
                            params["lin_w"], params["lin_b"].reshape(1, C))
    return out


# speedup vs baseline: 19.3840x; 1.3447x over previous
import functools

import jax
import jax.numpy as jnp
from jax import lax
from jax.experimental import pallas as pl
from jax.experimental.pallas import tpu as pltpu
from jax.experimental.pallas import tpu_sc as plsc

N = 10000
E = 320000
D = 128
H = 128
C = 40
NUM_LAYERS = 3

_CHUNK = 128
_NCHUNK = E // _CHUNK
_NW = 32
_RPT = 624
_ZCH = 104
_TAIL = 16
_TAIL_BASE = 16 * _RPT

_BLK = 1000
_GRID = N // _BLK


def _edge_range(w):
    lo = (w * _NCHUNK) // _NW
    hi = ((w + 1) * _NCHUNK) // _NW
    return lo, hi


@functools.cache
def _get_sc_degree():
    return functools.partial(
        pl.kernel,
        out_type=jax.ShapeDtypeStruct((2, N, H), jnp.float32),
        mesh=plsc.VectorSubcoreMesh(core_axis_name="c", subcore_axis_name="s"),
        scratch_types=[
            pltpu.VMEM((_CHUNK,), jnp.int32),
            pltpu.VMEM((_CHUNK, H), jnp.float32),
            pltpu.VMEM((_ZCH, H), jnp.float32),
            pltpu.VMEM_SHARED((N, H), jnp.float32),
        ],
    )(_sc_degree_body)


def _sc_degree_body(dst_hbm, ones_hbm, zeros_hbm, out_hbm, dst_v, ones_v, zero_v, acc):
    c = lax.axis_index("c")
    s = lax.axis_index("s")
    pltpu.sync_copy(zeros_hbm, zero_v)
    pltpu.sync_copy(ones_hbm, ones_v)
    for j in range(_RPT // _ZCH):
        pltpu.sync_copy(zero_v, acc.at[pl.ds(s * _RPT + j * _ZCH, _ZCH)])

    @pl.when(s == 15)
    def _zero_tail():
        pltpu.sync_copy(zero_v.at[pl.ds(0, _TAIL)], acc.at[pl.ds(_TAIL_BASE, _TAIL)])

    plsc.subcore_barrier()
    w = s * 2 + c
    lo, hi = _edge_range(w)

    def body(i, carry):
        pltpu.sync_copy(dst_hbm.at[pl.ds(i * _CHUNK, _CHUNK)], dst_v)
        pltpu.sync_copy(ones_v, acc.at[dst_v], add=True)
        return carry

    lax.fori_loop(lo, hi, body, 0)
    plsc.subcore_barrier()
    pltpu.sync_copy(acc.at[pl.ds(s * _RPT, _RPT)],
                    out_hbm.at[c, pl.ds(s * _RPT, _RPT)])

    @pl.when(s == 15)
    def _copy_tail():
        pltpu.sync_copy(acc.at[pl.ds(_TAIL_BASE, _TAIL)],
                        out_hbm.at[c, pl.ds(_TAIL_BASE, _TAIL)])


@functools.cache
def _get_sc_aggregate():
    return functools.partial(
        pl.kernel,
        out_type=jax.ShapeDtypeStruct((2, N, H), jnp.float32),
        mesh=plsc.VectorSubcoreMesh(core_axis_name="c", subcore_axis_name="s"),
        scratch_types=[
            pltpu.VMEM((_CHUNK,), jnp.int32),
            pltpu.VMEM((_CHUNK,), jnp.int32),
            pltpu.VMEM((_CHUNK,), jnp.int32),
            pltpu.VMEM((_CHUNK,), jnp.int32),
            pltpu.VMEM((_CHUNK,), jnp.int32),
            pltpu.VMEM((_CHUNK,), jnp.int32),
            pltpu.VMEM((_CHUNK,), jnp.int32),
            pltpu.VMEM((_CHUNK,), jnp.int32),
            pltpu.VMEM((_CHUNK, H), jnp.float32),
            pltpu.VMEM((_CHUNK, H), jnp.float32),
            pltpu.VMEM((_ZCH, H), jnp.float32),
            pltpu.VMEM_SHARED((N, H), jnp.float32),
            pltpu.SemaphoreType.DMA,
            pltpu.SemaphoreType.DMA,
            pltpu.SemaphoreType.DMA,
            pltpu.SemaphoreType.DMA,
            pltpu.SemaphoreType.DMA,
            pltpu.SemaphoreType.DMA,
        ],
    )(_sc_aggregate_body)


def _sc_aggregate_body(hwp_hbm, src_hbm, dst_hbm, zeros_hbm, out_hbm,
                       src0, src1, src2, src3, dst0, dst1, dst2, dst3,
                       rows0, rows1, zero_v, acc,
                       sem_i0, sem_i1, sem_i2, sem_i3, sem_g0, sem_g1):
    c = lax.axis_index("c")
    s = lax.axis_index("s")
    src_v = (src0, src1, src2, src3)
    dst_v = (dst0, dst1, dst2, dst3)
    rows_v = (rows0, rows1)
    sem_i = (sem_i0, sem_i1, sem_i2, sem_i3)
    sem_g = (sem_g0, sem_g1)

    pltpu.sync_copy(zeros_hbm, zero_v)
    for j in range(_RPT // _ZCH):
        pltpu.sync_copy(zero_v, acc.at[pl.ds(s * _RPT + j * _ZCH, _ZCH)])

    @pl.when(s == 15)
    def _zero_tail():
        pltpu.sync_copy(zero_v.at[pl.ds(0, _TAIL)], acc.at[pl.ds(_TAIL_BASE, _TAIL)])

    plsc.subcore_barrier()
    w = s * 2 + c
    lo, hi = _edge_range(w)

    def load_idx(i, j):
        pltpu.async_copy(src_hbm.at[pl.ds(i * _CHUNK, _CHUNK)], src_v[j], sem_i[j])
        pltpu.async_copy(dst_hbm.at[pl.ds(i * _CHUNK, _CHUNK)], dst_v[j], sem_i[j])

    def wait_idx(j):
        pltpu.make_async_copy(src_hbm.at[pl.ds(0, _CHUNK)], src_v[j], sem_i[j]).wait()
        pltpu.make_async_copy(dst_hbm.at[pl.ds(0, _CHUNK)], dst_v[j], sem_i[j]).wait()

    def start_gather(j, b):
        pltpu.async_copy(hwp_hbm.at[src_v[j]], rows_v[b], sem_g[b])

    def wait_gather(b):
        pltpu.make_async_copy(hwp_hbm.at[pl.ds(0, _CHUNK)], rows_v[b], sem_g[b]).wait()

    load_idx(lo, 0)
    load_idx(lo + 1, 1)
    load_idx(lo + 2, 2)
    wait_idx(0)
    start_gather(0, 0)

    def step(i, j):
        b = j % 2
        nb = (j + 1) % 2
        nj = (j + 1) % 4

        @pl.when(i + 1 < hi)
        def _next_gather():
            wait_idx(nj)
            start_gather(nj, nb)

        wait_gather(b)
        pltpu.sync_copy(rows_v[b], acc.at[dst_v[j]], add=True)

        @pl.when(i + 3 < hi)
        def _prefetch_idx():
            load_idx(i + 3, (j + 3) % 4)

    def quad(k, carry):
        i0 = lo + 4 * k
        for j in range(4):
            step(i0 + j, j)
        return carry

    nq = (hi - lo) // 4
    lax.fori_loop(0, nq, quad, 0)
    rem = (hi - lo) - 4 * nq
    for j in range(3):
        @pl.when(j < rem)
        def _tail_step(j=j):
            step(lo + 4 * nq + j, j)

    plsc.subcore_barrier()
    pltpu.sync_copy(acc.at[pl.ds(s * _RPT, _RPT)],
                    out_hbm.at[c, pl.ds(s * _RPT, _RPT)])

    @pl.when(s == 15)
    def _copy_tail():
        pltpu.sync_copy(acc.at[pl.ds(_TAIL_BASE, _TAIL)],
                        out_hbm.at[c, pl.ds(_TAIL_BASE, _TAIL)])


def _dinv_block(d0, d1):
    deg = 1.0 + d0[:, 0:1] + d1[:, 0:1]
    return lax.rsqrt(deg)


def _a0_body(la_ref, lb_ref, x1_ref, x2_ref, w1_ref, b1_ref, w2_ref,
             d0_ref, d1_ref, out_ref):
    alpha = jnp.exp(la_ref[0, 0])
    beta = jnp.exp(lb_ref[0, 0])
    h = alpha * x1_ref[...] + beta * x2_ref[...]
    m = jnp.maximum(jnp.dot(h, w1_ref[...], preferred_element_type=jnp.float32)
                    + b1_ref[...], 0.0)
    hw = jnp.dot(m, w2_ref[...], preferred_element_type=jnp.float32)
    out_ref[...] = hw * _dinv_block(d0_ref[...], d1_ref[...])


def _a_body(h_ref, w1_ref, b1_ref, w2_ref, d0_ref, d1_ref, out_ref):
    m = jnp.maximum(jnp.dot(h_ref[...], w1_ref[...],
                            preferred_element_type=jnp.float32) + b1_ref[...], 0.0)
    hw = jnp.dot(m, w2_ref[...], preferred_element_type=jnp.float32)
    out_ref[...] = hw * _dinv_block(d0_ref[...], d1_ref[...])


def _b1_body(a0_ref, a1_ref, hwp_ref, d0_ref, d1_ref, cb_ref,
             t_ref, s_ref, ss_ref):
    dinv = _dinv_block(d0_ref[...], d1_ref[...])
    t = dinv * (a0_ref[...] + a1_ref[...] + hwp_ref[...]) + cb_ref[...]
    t_ref[...] = t
    s_ref[...] = jnp.sum(t, axis=0, keepdims=True)[None]
    ss_ref[...] = jnp.sum(t * t, axis=0, keepdims=True)[None]


def _b2_body(t_ref, s_ref, ss_ref, g_ref, b_ref, out_ref):
    ssum = jnp.sum(s_ref[...], axis=0)
    sqsum = jnp.sum(ss_ref[...], axis=0)
    mean = ssum * (1.0 / N)
    var = sqsum * (1.0 / N) - mean * mean
    y = (t_ref[...] - mean) / jnp.sqrt(var + 1e-5) * g_ref[...] + b_ref[...]
    out_ref[...] = jnp.maximum(y, 0.0)


def _blast_body(a0_ref, a1_ref, hwp_ref, d0_ref, d1_ref, cb_ref,
                lw_ref, lb_ref, out_ref):
    dinv = _dinv_block(d0_ref[...], d1_ref[...])
    t = dinv * (a0_ref[...] + a1_ref[...] + hwp_ref[...]) + cb_ref[...]
    z = jnp.dot(t, lw_ref[...], preferred_element_type=jnp.float32) + lb_ref[...]
    z = z - jnp.max(z, axis=-1, keepdims=True)
    out_ref[...] = z - jnp.log(jnp.sum(jnp.exp(z), axis=-1, keepdims=True))


def _row_spec(width):
    return pl.BlockSpec((_BLK, width), lambda i: (i, 0))


def _full_spec(shape):
    return pl.BlockSpec(shape, lambda i: tuple(0 for _ in shape))


_SMEM_SPEC = pl.BlockSpec(memory_space=pltpu.SMEM)


def _tc_a0(la, lb, x1, x2, w1, b1, w2, d0, d1):
    return pl.pallas_call(
        _a0_body,
        grid=(_GRID,),
        in_specs=[_SMEM_SPEC, _SMEM_SPEC, _row_spec(D), _row_spec(D),
                  _full_spec((D, H)), _full_spec((1, H)), _full_spec((H, H)),
                  _row_spec(H), _row_spec(H)],
        out_specs=_row_spec(H),
        out_shape=jax.ShapeDtypeStruct((N, H), jnp.float32),
    )(la, lb, x1, x2, w1, b1, w2, d0, d1)


def _tc_a(h, w1, b1, w2, d0, d1):
    return pl.pallas_call(
        _a_body,
        grid=(_GRID,),
        in_specs=[_row_spec(H), _full_spec((H, H)), _full_spec((1, H)),
                  _full_spec((H, H)), _row_spec(H), _row_spec(H)],
        out_specs=_row_spec(H),
        out_shape=jax.ShapeDtypeStruct((N, H), jnp.float32),
    )(h, w1, b1, w2, d0, d1)


def _tc_b1(a0, a1, hwp, d0, d1, cb):
    return pl.pallas_call(
        _b1_body,
        grid=(_GRID,),
        in_specs=[_row_spec(H), _row_spec(H), _row_spec(H),
                  _row_spec(H), _row_spec(H), _full_spec((1, H))],
        out_specs=[_row_spec(H),
                   pl.BlockSpec((1, 1, H), lambda i: (i, 0, 0)),
                   pl.BlockSpec((1, 1, H), lambda i: (i, 0, 0))],
        out_shape=[jax.ShapeDtypeStruct((N, H), jnp.float32),
                   jax.ShapeDtypeStruct((_GRID, 1, H), jnp.float32),
                   jax.ShapeDtypeStruct((_GRID, 1, H), jnp.float32)],
    )(a0, a1, hwp, d0, d1, cb)


def _tc_b2(t, s, ss, g, b):
    return pl.pallas_call(
        _b2_body,
        grid=(_GRID,),
        in_specs=[_row_spec(H), _full_spec((_GRID, 1, H)), _full_spec((_GRID, 1, H)),
                  _full_spec((1, H)), _full_spec((1, H))],
        out_specs=_row_spec(H),
        out_shape=jax.ShapeDtypeStruct((N, H), jnp.float32),
    )(t, s, ss, g, b)


def _tc_blast(a0, a1, hwp, d0, d1, cb, lw, lb):
    return pl.pallas_call(
        _blast_body,
        grid=(_GRID,),
        in_specs=[_row_spec(H), _row_spec(H), _row_spec(H),
                  _row_spec(H), _row_spec(H), _full_spec((1, H)),
                  _full_spec((H, C)), _full_spec((1, C))],
        out_specs=_row_spec(C),
        out_shape=jax.ShapeDtypeStruct((N, C), jnp.float32),
    )(a0, a1, hwp, d0, d1, cb, lw, lb)


def kernel(x, edge_index, params):
    src = edge_index[0]
    dst = edge_index[1]
    x1 = x[:, :D]
    x2 = x[:, D:]

    ones128 = jnp.ones((_CHUNK, H), jnp.float32)
    zeros128 = jnp.zeros((_ZCH, H), jnp.float32)

    degp = _get_sc_degree()(dst, ones128, zeros128)
    d0 = degp[0]
    d1 = degp[1]

    la = params["log_alpha"].reshape(1, 1)
    lb = params["log_beta"].reshape(1, 1)

    h = None
    out = None
    for i in range(NUM_LAYERS):
        w1 = params["mlp_w"][i]
        b1 = params["mlp_b"][i].reshape(1, H)
        w2 = params["conv_w"][i]
        cb = params["conv_b"][i].reshape(1, H)
        if i == 0:
            hwp = _tc_a0(la, lb, x1, x2, w1, b1, w2, d0, d1)
        else:
            hwp = _tc_a(h, w1, b1, w2, d0, d1)
        agg = _get_sc_aggregate()(hwp, src, dst, zeros128)
        if i < NUM_LAYERS - 1:
            t, s, ss = _tc_b1(agg[0], agg[1], hwp, d0, d1, cb)
            h = _tc_b2(t, s, ss,
                       params["bn_gamma"][i].reshape(1, H),
                       params["bn_beta"][i].reshape(1, H))
        else:
            out = _tc_blast(agg[0], agg[1], hwp, d0, d1, cb,
                            params["lin_w"], params["lin_b"].reshape(1, C))
    return out


# speedup vs baseline: 21.3492x; 1.1014x over previous
import functools

import jax
import jax.numpy as jnp
from jax import lax
from jax.experimental import pallas as pl
from jax.experimental.pallas import tpu as pltpu
from jax.experimental.pallas import tpu_sc as plsc

N = 10000
E = 320000
D = 128
H = 128
C = 40
NUM_LAYERS = 3

_CHUNK = 128
_NCHUNK = E // _CHUNK
_NW = 32
_RPT = 624
_ZCH = 104
_TAIL = 16
_TAIL_BASE = 16 * _RPT

_BLK = 1000
_GRID = N // _BLK


def _edge_range(w):
    lo = (w * _NCHUNK) // _NW
    hi = ((w + 1) * _NCHUNK) // _NW
    return lo, hi


@functools.cache
def _get_sc_degree():
    return functools.partial(
        pl.kernel,
        out_type=jax.ShapeDtypeStruct((2, N, H), jnp.float32),
        mesh=plsc.VectorSubcoreMesh(core_axis_name="c", subcore_axis_name="s"),
        scratch_types=[
            pltpu.VMEM((_CHUNK,), jnp.int32),
            pltpu.VMEM((_CHUNK,), jnp.int32),
            pltpu.VMEM((_CHUNK,), jnp.int32),
            pltpu.VMEM((_CHUNK,), jnp.int32),
            pltpu.VMEM((_CHUNK, H), jnp.float32),
            pltpu.VMEM((_ZCH, H), jnp.float32),
            pltpu.VMEM_SHARED((N, H), jnp.float32),
            pltpu.SemaphoreType.DMA,
            pltpu.SemaphoreType.DMA,
            pltpu.SemaphoreType.DMA,
            pltpu.SemaphoreType.DMA,
        ],
    )(_sc_degree_body)


def _sc_degree_body(dst_hbm, ones_hbm, zeros_hbm, out_hbm,
                    dst0, dst1, dst2, dst3, ones_v, zero_v, acc,
                    sem_i0, sem_i1, sem_i2, sem_i3):
    c = lax.axis_index("c")
    s = lax.axis_index("s")
    dst_v = (dst0, dst1, dst2, dst3)
    sem_i = (sem_i0, sem_i1, sem_i2, sem_i3)
    pltpu.sync_copy(zeros_hbm, zero_v)
    pltpu.sync_copy(ones_hbm, ones_v)
    for j in range(_RPT // _ZCH):
        pltpu.sync_copy(zero_v, acc.at[pl.ds(s * _RPT + j * _ZCH, _ZCH)])

    @pl.when(s == 15)
    def _zero_tail():
        pltpu.sync_copy(zero_v.at[pl.ds(0, _TAIL)], acc.at[pl.ds(_TAIL_BASE, _TAIL)])

    plsc.subcore_barrier()
    w = s * 2 + c
    lo, hi = _edge_range(w)

    def load_idx(i, j):
        pltpu.async_copy(dst_hbm.at[pl.ds(i * _CHUNK, _CHUNK)], dst_v[j], sem_i[j])

    def wait_idx(j):
        pltpu.make_async_copy(dst_hbm.at[pl.ds(0, _CHUNK)], dst_v[j], sem_i[j]).wait()

    load_idx(lo, 0)
    load_idx(lo + 1, 1)
    load_idx(lo + 2, 2)

    def step(i, j):
        wait_idx(j)
        pltpu.sync_copy(ones_v, acc.at[dst_v[j]], add=True)

        @pl.when(i + 3 < hi)
        def _prefetch_idx():
            load_idx(i + 3, (j + 3) % 4)

    def quad(k, carry):
        i0 = lo + 4 * k
        for j in range(4):
            step(i0 + j, j)
        return carry

    nq = (hi - lo) // 4
    lax.fori_loop(0, nq, quad, 0)
    rem = (hi - lo) - 4 * nq
    for j in range(3):
        @pl.when(j < rem)
        def _tail_step(j=j):
            step(lo + 4 * nq + j, j)

    plsc.subcore_barrier()
    pltpu.sync_copy(acc.at[pl.ds(s * _RPT, _RPT)],
                    out_hbm.at[c, pl.ds(s * _RPT, _RPT)])

    @pl.when(s == 15)
    def _copy_tail():
        pltpu.sync_copy(acc.at[pl.ds(_TAIL_BASE, _TAIL)],
                        out_hbm.at[c, pl.ds(_TAIL_BASE, _TAIL)])


@functools.cache
def _get_sc_aggregate():
    return functools.partial(
        pl.kernel,
        out_type=jax.ShapeDtypeStruct((2, N, H), jnp.float32),
        mesh=plsc.VectorSubcoreMesh(core_axis_name="c", subcore_axis_name="s"),
        scratch_types=[
            pltpu.VMEM((_CHUNK,), jnp.int32),
            pltpu.VMEM((_CHUNK,), jnp.int32),
            pltpu.VMEM((_CHUNK,), jnp.int32),
            pltpu.VMEM((_CHUNK,), jnp.int32),
            pltpu.VMEM((_CHUNK,), jnp.int32),
            pltpu.VMEM((_CHUNK,), jnp.int32),
            pltpu.VMEM((_CHUNK,), jnp.int32),
            pltpu.VMEM((_CHUNK,), jnp.int32),
            pltpu.VMEM((_CHUNK, H), jnp.float32),
            pltpu.VMEM((_CHUNK, H), jnp.float32),
            pltpu.VMEM((_ZCH, H), jnp.float32),
            pltpu.VMEM_SHARED((N, H), jnp.float32),
            pltpu.SemaphoreType.DMA,
            pltpu.SemaphoreType.DMA,
            pltpu.SemaphoreType.DMA,
            pltpu.SemaphoreType.DMA,
            pltpu.SemaphoreType.DMA,
            pltpu.SemaphoreType.DMA,
        ],
    )(_sc_aggregate_body)


def _sc_aggregate_body(hwp_hbm, src_hbm, dst_hbm, zeros_hbm, out_hbm,
                       src0, src1, src2, src3, dst0, dst1, dst2, dst3,
                       rows0, rows1, zero_v, acc,
                       sem_i0, sem_i1, sem_i2, sem_i3, sem_g0, sem_g1):
    c = lax.axis_index("c")
    s = lax.axis_index("s")
    src_v = (src0, src1, src2, src3)
    dst_v = (dst0, dst1, dst2, dst3)
    rows_v = (rows0, rows1)
    sem_i = (sem_i0, sem_i1, sem_i2, sem_i3)
    sem_g = (sem_g0, sem_g1)

    pltpu.sync_copy(zeros_hbm, zero_v)
    for j in range(_RPT // _ZCH):
        pltpu.sync_copy(zero_v, acc.at[pl.ds(s * _RPT + j * _ZCH, _ZCH)])

    @pl.when(s == 15)
    def _zero_tail():
        pltpu.sync_copy(zero_v.at[pl.ds(0, _TAIL)], acc.at[pl.ds(_TAIL_BASE, _TAIL)])

    plsc.subcore_barrier()
    w = s * 2 + c
    lo, hi = _edge_range(w)

    def load_idx(i, j):
        pltpu.async_copy(src_hbm.at[pl.ds(i * _CHUNK, _CHUNK)], src_v[j], sem_i[j])
        pltpu.async_copy(dst_hbm.at[pl.ds(i * _CHUNK, _CHUNK)], dst_v[j], sem_i[j])

    def wait_idx(j):
        pltpu.make_async_copy(src_hbm.at[pl.ds(0, _CHUNK)], src_v[j], sem_i[j]).wait()
        pltpu.make_async_copy(dst_hbm.at[pl.ds(0, _CHUNK)], dst_v[j], sem_i[j]).wait()

    def start_gather(j, b):
        pltpu.async_copy(hwp_hbm.at[src_v[j]], rows_v[b], sem_g[b])

    def wait_gather(b):
        pltpu.make_async_copy(hwp_hbm.at[pl.ds(0, _CHUNK)], rows_v[b], sem_g[b]).wait()

    load_idx(lo, 0)
    load_idx(lo + 1, 1)
    load_idx(lo + 2, 2)
    wait_idx(0)
    start_gather(0, 0)

    def step(i, j):
        b = j % 2
        nb = (j + 1) % 2
        nj = (j + 1) % 4

        @pl.when(i + 1 < hi)
        def _next_gather():
            wait_idx(nj)
            start_gather(nj, nb)

        wait_gather(b)
        pltpu.sync_copy(rows_v[b], acc.at[dst_v[j]], add=True)

        @pl.when(i + 3 < hi)
        def _prefetch_idx():
            load_idx(i + 3, (j + 3) % 4)

    def quad(k, carry):
        i0 = lo + 4 * k
        for j in range(4):
            step(i0 + j, j)
        return carry

    nq = (hi - lo) // 4
    lax.fori_loop(0, nq, quad, 0)
    rem = (hi - lo) - 4 * nq
    for j in range(3):
        @pl.when(j < rem)
        def _tail_step(j=j):
            step(lo + 4 * nq + j, j)

    plsc.subcore_barrier()
    pltpu.sync_copy(acc.at[pl.ds(s * _RPT, _RPT)],
                    out_hbm.at[c, pl.ds(s * _RPT, _RPT)])

    @pl.when(s == 15)
    def _copy_tail():
        pltpu.sync_copy(acc.at[pl.ds(_TAIL_BASE, _TAIL)],
                        out_hbm.at[c, pl.ds(_TAIL_BASE, _TAIL)])


def _dinv_block(d0, d1):
    deg = 1.0 + d0[:, 0:1] + d1[:, 0:1]
    return lax.rsqrt(deg)


def _a0raw_body(la_ref, lb_ref, x1_ref, x2_ref, w1_ref, b1_ref, w2_ref, out_ref):
    alpha = jnp.exp(la_ref[0, 0])
    beta = jnp.exp(lb_ref[0, 0])
    h = alpha * x1_ref[...] + beta * x2_ref[...]
    m = jnp.maximum(jnp.dot(h, w1_ref[...], preferred_element_type=jnp.float32)
                    + b1_ref[...], 0.0)
    out_ref[...] = jnp.dot(m, w2_ref[...], preferred_element_type=jnp.float32)


def _scale_body(hw_ref, d0_ref, d1_ref, out_ref):
    out_ref[...] = hw_ref[...] * _dinv_block(d0_ref[...], d1_ref[...])


def _b1_body(a0_ref, a1_ref, hwp_ref, d0_ref, d1_ref, cb_ref,
             t_ref, s_ref, ss_ref):
    dinv = _dinv_block(d0_ref[...], d1_ref[...])
    t = dinv * (a0_ref[...] + a1_ref[...] + hwp_ref[...]) + cb_ref[...]
    t_ref[...] = t
    s_ref[...] = jnp.sum(t, axis=0, keepdims=True)[None]
    ss_ref[...] = jnp.sum(t * t, axis=0, keepdims=True)[None]


def _b2a_body(t_ref, s_ref, ss_ref, g_ref, b_ref, w1_ref, b1_ref, w2_ref,
              d0_ref, d1_ref, out_ref):
    ssum = jnp.sum(s_ref[...], axis=0)
    sqsum = jnp.sum(ss_ref[...], axis=0)
    mean = ssum * (1.0 / N)
    var = sqsum * (1.0 / N) - mean * mean
    y = (t_ref[...] - mean) / jnp.sqrt(var + 1e-5) * g_ref[...] + b_ref[...]
    h = jnp.maximum(y, 0.0)
    m = jnp.maximum(jnp.dot(h, w1_ref[...], preferred_element_type=jnp.float32)
                    + b1_ref[...], 0.0)
    hw = jnp.dot(m, w2_ref[...], preferred_element_type=jnp.float32)
    out_ref[...] = hw * _dinv_block(d0_ref[...], d1_ref[...])


def _blast_body(a0_ref, a1_ref, hwp_ref, d0_ref, d1_ref, cb_ref,
                lw_ref, lb_ref, out_ref):
    dinv = _dinv_block(d0_ref[...], d1_ref[...])
    t = dinv * (a0_ref[...] + a1_ref[...] + hwp_ref[...]) + cb_ref[...]
    z = jnp.dot(t, lw_ref[...], preferred_element_type=jnp.float32) + lb_ref[...]
    z = z - jnp.max(z, axis=-1, keepdims=True)
    out_ref[...] = z - jnp.log(jnp.sum(jnp.exp(z), axis=-1, keepdims=True))


def _row_spec(width):
    return pl.BlockSpec((_BLK, width), lambda i: (i, 0))


def _full_spec(shape):
    return pl.BlockSpec(shape, lambda i: tuple(0 for _ in shape))


_SMEM_SPEC = pl.BlockSpec(memory_space=pltpu.SMEM)


def _tc_a0raw(la, lb, x1, x2, w1, b1, w2):
    return pl.pallas_call(
        _a0raw_body,
        grid=(_GRID,),
        in_specs=[_SMEM_SPEC, _SMEM_SPEC, _row_spec(D), _row_spec(D),
                  _full_spec((D, H)), _full_spec((1, H)), _full_spec((H, H))],
        out_specs=_row_spec(H),
        out_shape=jax.ShapeDtypeStruct((N, H), jnp.float32),
    )(la, lb, x1, x2, w1, b1, w2)


def _tc_scale(hw, d0, d1):
    return pl.pallas_call(
        _scale_body,
        grid=(_GRID,),
        in_specs=[_row_spec(H), _row_spec(H), _row_spec(H)],
        out_specs=_row_spec(H),
        out_shape=jax.ShapeDtypeStruct((N, H), jnp.float32),
    )(hw, d0, d1)


def _tc_b1(a0, a1, hwp, d0, d1, cb):
    return pl.pallas_call(
        _b1_body,
        grid=(_GRID,),
        in_specs=[_row_spec(H), _row_spec(H), _row_spec(H),
                  _row_spec(H), _row_spec(H), _full_spec((1, H))],
        out_specs=[_row_spec(H),
                   pl.BlockSpec((1, 1, H), lambda i: (i, 0, 0)),
                   pl.BlockSpec((1, 1, H), lambda i: (i, 0, 0))],
        out_shape=[jax.ShapeDtypeStruct((N, H), jnp.float32),
                   jax.ShapeDtypeStruct((_GRID, 1, H), jnp.float32),
                   jax.ShapeDtypeStruct((_GRID, 1, H), jnp.float32)],
    )(a0, a1, hwp, d0, d1, cb)


def _tc_b2a(t, s, ss, g, b, w1, b1, w2, d0, d1):
    return pl.pallas_call(
        _b2a_body,
        grid=(_GRID,),
        in_specs=[_row_spec(H), _full_spec((_GRID, 1, H)), _full_spec((_GRID, 1, H)),
                  _full_spec((1, H)), _full_spec((1, H)),
                  _full_spec((H, H)), _full_spec((1, H)), _full_spec((H, H)),
                  _row_spec(H), _row_spec(H)],
        out_specs=_row_spec(H),
        out_shape=jax.ShapeDtypeStruct((N, H), jnp.float32),
    )(t, s, ss, g, b, w1, b1, w2, d0, d1)


def _tc_blast(a0, a1, hwp, d0, d1, cb, lw, lb):
    return pl.pallas_call(
        _blast_body,
        grid=(_GRID,),
        in_specs=[_row_spec(H), _row_spec(H), _row_spec(H),
                  _row_spec(H), _row_spec(H), _full_spec((1, H)),
                  _full_spec((H, C)), _full_spec((1, C))],
        out_specs=_row_spec(C),
        out_shape=jax.ShapeDtypeStruct((N, C), jnp.float32),
    )(a0, a1, hwp, d0, d1, cb, lw, lb)


def kernel(x, edge_index, params):
    src = edge_index[0]
    dst = edge_index[1]
    x1 = x[:, :D]
    x2 = x[:, D:]

    ones128 = jnp.ones((_CHUNK, H), jnp.float32)
    zeros128 = jnp.zeros((_ZCH, H), jnp.float32)

    degp = _get_sc_degree()(dst, ones128, zeros128)
    d0 = degp[0]
    d1 = degp[1]

    la = params["log_alpha"].reshape(1, 1)
    lb = params["log_beta"].reshape(1, 1)

    out = None
    hwp = None
    for i in range(NUM_LAYERS):
        w1 = params["mlp_w"][i]
        b1 = params["mlp_b"][i].reshape(1, H)
        w2 = params["conv_w"][i]
        cb = params["conv_b"][i].reshape(1, H)
        if i == 0:
            hw0 = _tc_a0raw(la, lb, x1, x2, w1, b1, w2)
            hwp = _tc_scale(hw0, d0, d1)
        agg = _get_sc_aggregate()(hwp, src, dst, zeros128)
        if i < NUM_LAYERS - 1:
            t, s, ss = _tc_b1(agg[0], agg[1], hwp, d0, d1, cb)
            hwp = _tc_b2a(t, s, ss,
                          params["bn_gamma"][i].reshape(1, H),
                          params["bn_beta"][i].reshape(1, H),
                          params["mlp_w"][i + 1],
                          params["mlp_b"][i + 1].reshape(1, H),
                          params["conv_w"][i + 1], d0, d1)
        else:
            out = _tc_blast(agg[0], agg[1], hwp, d0, d1, cb,
                            params["lin_w"], params["lin_b"].reshape(1, C))
    return out


# speedup vs baseline: 23.3239x; 1.0925x over previous
import functools

import jax
import jax.numpy as jnp
from jax import lax
from jax.experimental import pallas as pl
from jax.experimental.pallas import tpu as pltpu
from jax.experimental.pallas import tpu_sc as plsc

N = 10000
E = 320000
D = 128
H = 128
C = 40
NUM_LAYERS = 3

_CHUNK = 128
_NCHUNK = E // _CHUNK
_NW = 32
_RPT = 624
_ZCH = 104
_TAIL = 16
_TAIL_BASE = 16 * _RPT

_BLK = 2000
_GRID = N // _BLK


def _edge_range(w):
    lo = (w * _NCHUNK) // _NW
    hi = ((w + 1) * _NCHUNK) // _NW
    return lo, hi


@functools.cache
def _get_sc_degree():
    return functools.partial(
        pl.kernel,
        out_type=jax.ShapeDtypeStruct((2, N, H), jnp.float32),
        mesh=plsc.VectorSubcoreMesh(core_axis_name="c", subcore_axis_name="s"),
        scratch_types=[
            pltpu.VMEM((_CHUNK,), jnp.int32),
            pltpu.VMEM((_CHUNK,), jnp.int32),
            pltpu.VMEM((_CHUNK,), jnp.int32),
            pltpu.VMEM((_CHUNK,), jnp.int32),
            pltpu.VMEM((_CHUNK, H), jnp.float32),
            pltpu.VMEM((_ZCH, H), jnp.float32),
            pltpu.VMEM_SHARED((N, H), jnp.float32),
            pltpu.SemaphoreType.DMA,
            pltpu.SemaphoreType.DMA,
            pltpu.SemaphoreType.DMA,
            pltpu.SemaphoreType.DMA,
        ],
    )(_sc_degree_body)


def _sc_degree_body(dst_hbm, ones_hbm, zeros_hbm, out_hbm,
                    dst0, dst1, dst2, dst3, ones_v, zero_v, acc,
                    sem_i0, sem_i1, sem_i2, sem_i3):
    c = lax.axis_index("c")
    s = lax.axis_index("s")
    dst_v = (dst0, dst1, dst2, dst3)
    sem_i = (sem_i0, sem_i1, sem_i2, sem_i3)
    pltpu.sync_copy(zeros_hbm, zero_v)
    pltpu.sync_copy(ones_hbm, ones_v)
    for j in range(_RPT // _ZCH):
        pltpu.sync_copy(zero_v, acc.at[pl.ds(s * _RPT + j * _ZCH, _ZCH)])

    @pl.when(s == 15)
    def _zero_tail():
        pltpu.sync_copy(zero_v.at[pl.ds(0, _TAIL)], acc.at[pl.ds(_TAIL_BASE, _TAIL)])

    plsc.subcore_barrier()
    w = s * 2 + c
    lo, hi = _edge_range(w)

    def load_idx(i, j):
        pltpu.async_copy(dst_hbm.at[pl.ds(i * _CHUNK, _CHUNK)], dst_v[j], sem_i[j])

    def wait_idx(j):
        pltpu.make_async_copy(dst_hbm.at[pl.ds(0, _CHUNK)], dst_v[j], sem_i[j]).wait()

    load_idx(lo, 0)
    load_idx(lo + 1, 1)
    load_idx(lo + 2, 2)

    def step(i, j):
        wait_idx(j)
        pltpu.sync_copy(ones_v, acc.at[dst_v[j]], add=True)

        @pl.when(i + 3 < hi)
        def _prefetch_idx():
            load_idx(i + 3, (j + 3) % 4)

    def quad(k, carry):
        i0 = lo + 4 * k
        for j in range(4):
            step(i0 + j, j)
        return carry

    nq = (hi - lo) // 4
    lax.fori_loop(0, nq, quad, 0)
    rem = (hi - lo) - 4 * nq
    for j in range(3):
        @pl.when(j < rem)
        def _tail_step(j=j):
            step(lo + 4 * nq + j, j)

    plsc.subcore_barrier()
    pltpu.sync_copy(acc.at[pl.ds(s * _RPT, _RPT)],
                    out_hbm.at[c, pl.ds(s * _RPT, _RPT)])

    @pl.when(s == 15)
    def _copy_tail():
        pltpu.sync_copy(acc.at[pl.ds(_TAIL_BASE, _TAIL)],
                        out_hbm.at[c, pl.ds(_TAIL_BASE, _TAIL)])


@functools.cache
def _get_sc_aggregate():
    return functools.partial(
        pl.kernel,
        out_type=jax.ShapeDtypeStruct((2, N, H), jnp.float32),
        mesh=plsc.VectorSubcoreMesh(core_axis_name="c", subcore_axis_name="s"),
        scratch_types=[
            pltpu.VMEM((_CHUNK,), jnp.int32),
            pltpu.VMEM((_CHUNK,), jnp.int32),
            pltpu.VMEM((_CHUNK,), jnp.int32),
            pltpu.VMEM((_CHUNK,), jnp.int32),
            pltpu.VMEM((_CHUNK,), jnp.int32),
            pltpu.VMEM((_CHUNK,), jnp.int32),
            pltpu.VMEM((_CHUNK,), jnp.int32),
            pltpu.VMEM((_CHUNK,), jnp.int32),
            pltpu.VMEM((_CHUNK, H), jnp.float32),
            pltpu.VMEM((_CHUNK, H), jnp.float32),
            pltpu.VMEM((_ZCH, H), jnp.float32),
            pltpu.VMEM_SHARED((N, H), jnp.float32),
            pltpu.SemaphoreType.DMA,
            pltpu.SemaphoreType.DMA,
            pltpu.SemaphoreType.DMA,
            pltpu.SemaphoreType.DMA,
            pltpu.SemaphoreType.DMA,
            pltpu.SemaphoreType.DMA,
        ],
    )(_sc_aggregate_body)


def _sc_aggregate_body(hwp_hbm, src_hbm, dst_hbm, zeros_hbm, out_hbm,
                       src0, src1, src2, src3, dst0, dst1, dst2, dst3,
                       rows0, rows1, zero_v, acc,
                       sem_i0, sem_i1, sem_i2, sem_i3, sem_g0, sem_g1):
    c = lax.axis_index("c")
    s = lax.axis_index("s")
    src_v = (src0, src1, src2, src3)
    dst_v = (dst0, dst1, dst2, dst3)
    rows_v = (rows0, rows1)
    sem_i = (sem_i0, sem_i1, sem_i2, sem_i3)
    sem_g = (sem_g0, sem_g1)

    pltpu.sync_copy(zeros_hbm, zero_v)
    for j in range(_RPT // _ZCH):
        pltpu.sync_copy(zero_v, acc.at[pl.ds(s * _RPT + j * _ZCH, _ZCH)])

    @pl.when(s == 15)
    def _zero_tail():
        pltpu.sync_copy(zero_v.at[pl.ds(0, _TAIL)], acc.at[pl.ds(_TAIL_BASE, _TAIL)])

    plsc.subcore_barrier()
    w = s * 2 + c
    lo, hi = _edge_range(w)

    def load_idx(i, j):
        pltpu.async_copy(src_hbm.at[pl.ds(i * _CHUNK, _CHUNK)], src_v[j], sem_i[j])
        pltpu.async_copy(dst_hbm.at[pl.ds(i * _CHUNK, _CHUNK)], dst_v[j], sem_i[j])

    def wait_idx(j):
        pltpu.make_async_copy(src_hbm.at[pl.ds(0, _CHUNK)], src_v[j], sem_i[j]).wait()
        pltpu.make_async_copy(dst_hbm.at[pl.ds(0, _CHUNK)], dst_v[j], sem_i[j]).wait()

    def start_gather(j, b):
        pltpu.async_copy(hwp_hbm.at[src_v[j]], rows_v[b], sem_g[b])

    def wait_gather(b):
        pltpu.make_async_copy(hwp_hbm.at[pl.ds(0, _CHUNK)], rows_v[b], sem_g[b]).wait()

    load_idx(lo, 0)
    load_idx(lo + 1, 1)
    load_idx(lo + 2, 2)
    wait_idx(0)
    start_gather(0, 0)

    def step(i, j):
        b = j % 2
        nb = (j + 1) % 2
        nj = (j + 1) % 4

        @pl.when(i + 1 < hi)
        def _next_gather():
            wait_idx(nj)
            start_gather(nj, nb)

        wait_gather(b)
        pltpu.sync_copy(rows_v[b], acc.at[dst_v[j]], add=True)

        @pl.when(i + 3 < hi)
        def _prefetch_idx():
            load_idx(i + 3, (j + 3) % 4)

    def quad(k, carry):
        i0 = lo + 4 * k
        for j in range(4):
            step(i0 + j, j)
        return carry

    nq = (hi - lo) // 4
    lax.fori_loop(0, nq, quad, 0)
    rem = (hi - lo) - 4 * nq
    for j in range(3):
        @pl.when(j < rem)
        def _tail_step(j=j):
            step(lo + 4 * nq + j, j)

    plsc.subcore_barrier()
    pltpu.sync_copy(acc.at[pl.ds(s * _RPT, _RPT)],
                    out_hbm.at[c, pl.ds(s * _RPT, _RPT)])

    @pl.when(s == 15)
    def _copy_tail():
        pltpu.sync_copy(acc.at[pl.ds(_TAIL_BASE, _TAIL)],
                        out_hbm.at[c, pl.ds(_TAIL_BASE, _TAIL)])


def _dinv_block(d):
    deg = 1.0 + d[0, :, 0:1] + d[1, :, 0:1]
    return lax.rsqrt(deg)


def _a0raw_body(la_ref, lb_ref, x_ref, w1_ref, b1_ref, w2_ref, out_ref):
    alpha = jnp.exp(la_ref[0, 0])
    beta = jnp.exp(lb_ref[0, 0])
    h = alpha * x_ref[:, :D] + beta * x_ref[:, D:]
    m = jnp.maximum(jnp.dot(h, w1_ref[...], preferred_element_type=jnp.float32)
                    + b1_ref[...], 0.0)
    out_ref[...] = jnp.dot(m, w2_ref[...], preferred_element_type=jnp.float32)


def _scale_body(hw_ref, d_ref, out_ref):
    out_ref[...] = hw_ref[...] * _dinv_block(d_ref[...])


def _b1_body(a_ref, hwp_ref, d_ref, cb_ref, t_ref, s_ref, ss_ref):
    a = a_ref[...]
    dinv = _dinv_block(d_ref[...])
    t = dinv * (a[0] + a[1] + hwp_ref[...]) + cb_ref[...]
    t_ref[...] = t
    s_ref[...] = jnp.sum(t, axis=0, keepdims=True)[None]
    ss_ref[...] = jnp.sum(t * t, axis=0, keepdims=True)[None]


def _b2a_body(t_ref, s_ref, ss_ref, g_ref, b_ref, w1_ref, b1_ref, w2_ref,
              d_ref, out_ref):
    ssum = jnp.sum(s_ref[...], axis=0)
    sqsum = jnp.sum(ss_ref[...], axis=0)
    mean = ssum * (1.0 / N)
    var = sqsum * (1.0 / N) - mean * mean
    y = (t_ref[...] - mean) / jnp.sqrt(var + 1e-5) * g_ref[...] + b_ref[...]
    h = jnp.maximum(y, 0.0)
    m = jnp.maximum(jnp.dot(h, w1_ref[...], preferred_element_type=jnp.float32)
                    + b1_ref[...], 0.0)
    hw = jnp.dot(m, w2_ref[...], preferred_element_type=jnp.float32)
    out_ref[...] = hw * _dinv_block(d_ref[...])


def _blast_body(a_ref, hwp_ref, d_ref, cb_ref, lw_ref, lb_ref, out_ref):
    a = a_ref[...]
    dinv = _dinv_block(d_ref[...])
    t = dinv * (a[0] + a[1] + hwp_ref[...]) + cb_ref[...]
    z = jnp.dot(t, lw_ref[...], preferred_element_type=jnp.float32) + lb_ref[...]
    z = z - jnp.max(z, axis=-1, keepdims=True)
    out_ref[...] = z - jnp.log(jnp.sum(jnp.exp(z), axis=-1, keepdims=True))


def _row_spec(width):
    return pl.BlockSpec((_BLK, width), lambda i: (i, 0))


def _pair_spec():
    return pl.BlockSpec((2, _BLK, H), lambda i: (0, i, 0))


def _full_spec(shape):
    return pl.BlockSpec(shape, lambda i: tuple(0 for _ in shape))


_SMEM_SPEC = pl.BlockSpec(memory_space=pltpu.SMEM)


def _tc_a0raw(la, lb, x, w1, b1, w2):
    return pl.pallas_call(
        _a0raw_body,
        grid=(_GRID,),
        in_specs=[_SMEM_SPEC, _SMEM_SPEC, _row_spec(2 * D),
                  _full_spec((D, H)), _full_spec((1, H)), _full_spec((H, H))],
        out_specs=_row_spec(H),
        out_shape=jax.ShapeDtypeStruct((N, H), jnp.float32),
    )(la, lb, x, w1, b1, w2)


def _tc_scale(hw, degp):
    return pl.pallas_call(
        _scale_body,
        grid=(_GRID,),
        in_specs=[_row_spec(H), _pair_spec()],
        out_specs=_row_spec(H),
        out_shape=jax.ShapeDtypeStruct((N, H), jnp.float32),
    )(hw, degp)


def _tc_b1(agg, hwp, degp, cb):
    return pl.pallas_call(
        _b1_body,
        grid=(_GRID,),
        in_specs=[_pair_spec(), _row_spec(H), _pair_spec(), _full_spec((1, H))],
        out_specs=[_row_spec(H),
                   pl.BlockSpec((1, 1, H), lambda i: (i, 0, 0)),
                   pl.BlockSpec((1, 1, H), lambda i: (i, 0, 0))],
        out_shape=[jax.ShapeDtypeStruct((N, H), jnp.float32),
                   jax.ShapeDtypeStruct((_GRID, 1, H), jnp.float32),
                   jax.ShapeDtypeStruct((_GRID, 1, H), jnp.float32)],
    )(agg, hwp, degp, cb)


def _tc_b2a(t, s, ss, g, b, w1, b1, w2, degp):
    return pl.pallas_call(
        _b2a_body,
        grid=(_GRID,),
        in_specs=[_row_spec(H), _full_spec((_GRID, 1, H)), _full_spec((_GRID, 1, H)),
                  _full_spec((1, H)), _full_spec((1, H)),
                  _full_spec((H, H)), _full_spec((1, H)), _full_spec((H, H)),
                  _pair_spec()],
        out_specs=_row_spec(H),
        out_shape=jax.ShapeDtypeStruct((N, H), jnp.float32),
    )(t, s, ss, g, b, w1, b1, w2, degp)


def _tc_blast(agg, hwp, degp, cb, lw, lb):
    return pl.pallas_call(
        _blast_body,
        grid=(_GRID,),
        in_specs=[_pair_spec(), _row_spec(H), _pair_spec(), _full_spec((1, H)),
                  _full_spec((H, C)), _full_spec((1, C))],
        out_specs=_row_spec(C),
        out_shape=jax.ShapeDtypeStruct((N, C), jnp.float32),
    )(agg, hwp, degp, cb, lw, lb)


def kernel(x, edge_index, params):
    src = edge_index[0]
    dst = edge_index[1]

    ones128 = jnp.ones((_CHUNK, H), jnp.float32)
    zeros128 = jnp.zeros((_ZCH, H), jnp.float32)

    degp = _get_sc_degree()(dst, ones128, zeros128)

    la = params["log_alpha"].reshape(1, 1)
    lb = params["log_beta"].reshape(1, 1)

    out = None
    hwp = None
    for i in range(NUM_LAYERS):
        w1 = params["mlp_w"][i]
        b1 = params["mlp_b"][i].reshape(1, H)
        w2 = params["conv_w"][i]
        cb = params["conv_b"][i].reshape(1, H)
        if i == 0:
            hw0 = _tc_a0raw(la, lb, x, w1, b1, w2)
            hwp = _tc_scale(hw0, degp)
        agg = _get_sc_aggregate()(hwp, src, dst, zeros128)
        if i < NUM_LAYERS - 1:
            t, s, ss = _tc_b1(agg, hwp, degp, cb)
            hwp = _tc_b2a(t, s, ss,
                          params["bn_gamma"][i].reshape(1, H),
                          params["bn_beta"][i].reshape(1, H),
                          params["mlp_w"][i + 1],
                          params["mlp_b"][i + 1].reshape(1, H),
                          params["conv_w"][i + 1], degp)
        else:
            out = _tc_blast(agg, hwp, degp, cb,
                            params["lin_w"], params["lin_b"].reshape(1, C))
    return out


# speedup vs baseline: 24.1364x; 1.0348x over previous
import functools

import jax
import jax.numpy as jnp
from jax import lax
from jax.experimental import pallas as pl
from jax.experimental.pallas import tpu as pltpu
from jax.experimental.pallas import tpu_sc as plsc

N = 10000
E = 320000
D = 128
H = 128
C = 40
NUM_LAYERS = 3

_CHUNK = 128
_NCHUNK = E // _CHUNK
_NW = 32
_RPT = 624
_ZCH = 104
_TAIL = 16
_TAIL_BASE = 16 * _RPT

_BLK = 2000
_GRID = N // _BLK


def _edge_range(w):
    lo = (w * _NCHUNK) // _NW
    hi = ((w + 1) * _NCHUNK) // _NW
    return lo, hi


@functools.cache
def _get_sc_degree():
    return functools.partial(
        pl.kernel,
        out_type=jax.ShapeDtypeStruct((2, N, H), jnp.float32),
        mesh=plsc.VectorSubcoreMesh(core_axis_name="c", subcore_axis_name="s"),
        scratch_types=[
            pltpu.VMEM((_CHUNK,), jnp.int32),
            pltpu.VMEM((_CHUNK,), jnp.int32),
            pltpu.VMEM((_CHUNK,), jnp.int32),
            pltpu.VMEM((_CHUNK,), jnp.int32),
            pltpu.VMEM((_CHUNK, H), jnp.float32),
            pltpu.VMEM((_ZCH, H), jnp.float32),
            pltpu.VMEM_SHARED((N, H), jnp.float32),
            pltpu.SemaphoreType.DMA,
            pltpu.SemaphoreType.DMA,
            pltpu.SemaphoreType.DMA,
            pltpu.SemaphoreType.DMA,
        ],
    )(_sc_degree_body)


def _sc_degree_body(ei_hbm, ones_hbm, zeros_hbm, out_hbm,
                    dst0, dst1, dst2, dst3, ones_v, zero_v, acc,
                    sem_i0, sem_i1, sem_i2, sem_i3):
    c = lax.axis_index("c")
    s = lax.axis_index("s")
    dst_v = (dst0, dst1, dst2, dst3)
    sem_i = (sem_i0, sem_i1, sem_i2, sem_i3)
    pltpu.sync_copy(zeros_hbm, zero_v)
    pltpu.sync_copy(ones_hbm, ones_v)
    for j in range(_RPT // _ZCH):
        pltpu.sync_copy(zero_v, acc.at[pl.ds(s * _RPT + j * _ZCH, _ZCH)])

    @pl.when(s == 15)
    def _zero_tail():
        pltpu.sync_copy(zero_v.at[pl.ds(0, _TAIL)], acc.at[pl.ds(_TAIL_BASE, _TAIL)])

    plsc.subcore_barrier()
    w = s * 2 + c
    lo, hi = _edge_range(w)

    def load_idx(i, j):
        pltpu.async_copy(ei_hbm.at[1, i], dst_v[j], sem_i[j])

    def wait_idx(j):
        pltpu.make_async_copy(ei_hbm.at[1, 0], dst_v[j], sem_i[j]).wait()

    load_idx(lo, 0)
    load_idx(lo + 1, 1)
    load_idx(lo + 2, 2)

    def step(i, j):
        wait_idx(j)
        pltpu.sync_copy(ones_v, acc.at[dst_v[j]], add=True)

        @pl.when(i + 3 < hi)
        def _prefetch_idx():
            load_idx(i + 3, (j + 3) % 4)

    def quad(k, carry):
        i0 = lo + 4 * k
        for j in range(4):
            step(i0 + j, j)
        return carry

    nq = (hi - lo) // 4
    lax.fori_loop(0, nq, quad, 0)
    rem = (hi - lo) - 4 * nq
    for j in range(3):
        @pl.when(j < rem)
        def _tail_step(j=j):
            step(lo + 4 * nq + j, j)

    plsc.subcore_barrier()
    pltpu.sync_copy(acc.at[pl.ds(s * _RPT, _RPT)],
                    out_hbm.at[c, pl.ds(s * _RPT, _RPT)])

    @pl.when(s == 15)
    def _copy_tail():
        pltpu.sync_copy(acc.at[pl.ds(_TAIL_BASE, _TAIL)],
                        out_hbm.at[c, pl.ds(_TAIL_BASE, _TAIL)])


@functools.cache
def _get_sc_aggregate():
    return functools.partial(
        pl.kernel,
        out_type=jax.ShapeDtypeStruct((2, N, H), jnp.float32),
        mesh=plsc.VectorSubcoreMesh(core_axis_name="c", subcore_axis_name="s"),
        scratch_types=[
            pltpu.VMEM((_CHUNK,), jnp.int32),
            pltpu.VMEM((_CHUNK,), jnp.int32),
            pltpu.VMEM((_CHUNK,), jnp.int32),
            pltpu.VMEM((_CHUNK,), jnp.int32),
            pltpu.VMEM((_CHUNK,), jnp.int32),
            pltpu.VMEM((_CHUNK,), jnp.int32),
            pltpu.VMEM((_CHUNK,), jnp.int32),
            pltpu.VMEM((_CHUNK,), jnp.int32),
            pltpu.VMEM((_CHUNK, H), jnp.float32),
            pltpu.VMEM((_CHUNK, H), jnp.float32),
            pltpu.VMEM((_ZCH, H), jnp.float32),
            pltpu.VMEM_SHARED((N, H), jnp.float32),
            pltpu.SemaphoreType.DMA,
            pltpu.SemaphoreType.DMA,
            pltpu.SemaphoreType.DMA,
            pltpu.SemaphoreType.DMA,
            pltpu.SemaphoreType.DMA,
            pltpu.SemaphoreType.DMA,
        ],
    )(_sc_aggregate_body)


def _sc_aggregate_body(hwp_hbm, ei_hbm, zeros_hbm, out_hbm,
                       src0, src1, src2, src3, dst0, dst1, dst2, dst3,
                       rows0, rows1, zero_v, acc,
                       sem_i0, sem_i1, sem_i2, sem_i3, sem_g0, sem_g1):
    c = lax.axis_index("c")
    s = lax.axis_index("s")
    src_v = (src0, src1, src2, src3)
    dst_v = (dst0, dst1, dst2, dst3)
    rows_v = (rows0, rows1)
    sem_i = (sem_i0, sem_i1, sem_i2, sem_i3)
    sem_g = (sem_g0, sem_g1)

    pltpu.sync_copy(zeros_hbm, zero_v)
    for j in range(_RPT // _ZCH):
        pltpu.sync_copy(zero_v, acc.at[pl.ds(s * _RPT + j * _ZCH, _ZCH)])

    @pl.when(s == 15)
    def _zero_tail():
        pltpu.sync_copy(zero_v.at[pl.ds(0, _TAIL)], acc.at[pl.ds(_TAIL_BASE, _TAIL)])

    plsc.subcore_barrier()
    w = s * 2 + c
    lo, hi = _edge_range(w)

    def load_idx(i, j):
        pltpu.async_copy(ei_hbm.at[0, i], src_v[j], sem_i[j])
        pltpu.async_copy(ei_hbm.at[1, i], dst_v[j], sem_i[j])

    def wait_idx(j):
        pltpu.make_async_copy(ei_hbm.at[0, 0], src_v[j], sem_i[j]).wait()
        pltpu.make_async_copy(ei_hbm.at[1, 0], dst_v[j], sem_i[j]).wait()

    def start_gather(j, b):
        pltpu.async_copy(hwp_hbm.at[src_v[j]], rows_v[b], sem_g[b])

    def wait_gather(b):
        pltpu.make_async_copy(hwp_hbm.at[pl.ds(0, _CHUNK)], rows_v[b], sem_g[b]).wait()

    load_idx(lo, 0)
    load_idx(lo + 1, 1)
    load_idx(lo + 2, 2)
    wait_idx(0)
    start_gather(0, 0)

    def step(i, j):
        b = j % 2
        nb = (j + 1) % 2
        nj = (j + 1) % 4

        @pl.when(i + 1 < hi)
        def _next_gather():
            wait_idx(nj)
            start_gather(nj, nb)

        wait_gather(b)
        pltpu.sync_copy(rows_v[b], acc.at[dst_v[j]], add=True)

        @pl.when(i + 3 < hi)
        def _prefetch_idx():
            load_idx(i + 3, (j + 3) % 4)

    def quad(k, carry):
        i0 = lo + 4 * k
        for j in range(4):
            step(i0 + j, j)
        return carry

    nq = (hi - lo) // 4
    lax.fori_loop(0, nq, quad, 0)
    rem = (hi - lo) - 4 * nq
    for j in range(3):
        @pl.when(j < rem)
        def _tail_step(j=j):
            step(lo + 4 * nq + j, j)

    plsc.subcore_barrier()
    pltpu.sync_copy(acc.at[pl.ds(s * _RPT, _RPT)],
                    out_hbm.at[c, pl.ds(s * _RPT, _RPT)])

    @pl.when(s == 15)
    def _copy_tail():
        pltpu.sync_copy(acc.at[pl.ds(_TAIL_BASE, _TAIL)],
                        out_hbm.at[c, pl.ds(_TAIL_BASE, _TAIL)])


def _dinv_block(d):
    deg = 1.0 + d[0, :, 0:1] + d[1, :, 0:1]
    return lax.rsqrt(deg)


def _a0raw_body(la_ref, lb_ref, x_ref, w1_ref, b1_ref, w2_ref, out_ref):
    alpha = jnp.exp(la_ref[0, 0])
    beta = jnp.exp(lb_ref[0, 0])
    h = alpha * x_ref[:, :D] + beta * x_ref[:, D:]
    m = jnp.maximum(jnp.dot(h, w1_ref[...], preferred_element_type=jnp.float32)
                    + b1_ref[...], 0.0)
    out_ref[...] = jnp.dot(m, w2_ref[...], preferred_element_type=jnp.float32)


def _scale_body(hw_ref, d_ref, out_ref):
    out_ref[...] = hw_ref[...] * _dinv_block(d_ref[...])


def _bn_body(agg_ref, hwp_ref, d_ref, cb_ref, g_ref, b_ref,
             w1_ref, b1_ref, w2_ref, out_ref, t_sc, stat_sc):
    p = pl.program_id(0)
    i = pl.program_id(1)
    dinv = _dinv_block(d_ref[...])

    @pl.when(p == 0)
    def _phase_stats():
        a = agg_ref[...]
        t = dinv * (a[0] + a[1] + hwp_ref[...]) + cb_ref[...]
        t_sc[pl.ds(i * _BLK, _BLK), :] = t
        s = jnp.sum(t, axis=0, keepdims=True)
        ss = jnp.sum(t * t, axis=0, keepdims=True)

        @pl.when(i == 0)
        def _init():
            stat_sc[0:1, :] = s
            stat_sc[1:2, :] = ss

        @pl.when(i > 0)
        def _accum():
            stat_sc[0:1, :] = stat_sc[0:1, :] + s
            stat_sc[1:2, :] = stat_sc[1:2, :] + ss

    @pl.when(p == 1)
    def _phase_apply():
        mean = stat_sc[0:1, :] * (1.0 / N)
        var = stat_sc[1:2, :] * (1.0 / N) - mean * mean
        t = t_sc[pl.ds(i * _BLK, _BLK), :]
        y = (t - mean) / jnp.sqrt(var + 1e-5) * g_ref[...] + b_ref[...]
        h = jnp.maximum(y, 0.0)
        m = jnp.maximum(jnp.dot(h, w1_ref[...], preferred_element_type=jnp.float32)
                        + b1_ref[...], 0.0)
        hw = jnp.dot(m, w2_ref[...], preferred_element_type=jnp.float32)
        out_ref[...] = hw * dinv


def _blast_body(a_ref, hwp_ref, d_ref, cb_ref, lw_ref, lb_ref, out_ref):
    a = a_ref[...]
    dinv = _dinv_block(d_ref[...])
    t = dinv * (a[0] + a[1] + hwp_ref[...]) + cb_ref[...]
    z = jnp.dot(t, lw_ref[...], preferred_element_type=jnp.float32) + lb_ref[...]
    z = z - jnp.max(z, axis=-1, keepdims=True)
    out_ref[...] = z - jnp.log(jnp.sum(jnp.exp(z), axis=-1, keepdims=True))


def _row_spec(width):
    return pl.BlockSpec((_BLK, width), lambda i: (i, 0))


def _pair_spec():
    return pl.BlockSpec((2, _BLK, H), lambda i: (0, i, 0))


def _full_spec(shape):
    return pl.BlockSpec(shape, lambda i: tuple(0 for _ in shape))


_SMEM_SPEC = pl.BlockSpec(memory_space=pltpu.SMEM)


def _tc_a0raw(la, lb, x, w1, b1, w2):
    return pl.pallas_call(
        _a0raw_body,
        grid=(_GRID,),
        in_specs=[_SMEM_SPEC, _SMEM_SPEC, _row_spec(2 * D),
                  _full_spec((D, H)), _full_spec((1, H)), _full_spec((H, H))],
        out_specs=_row_spec(H),
        out_shape=jax.ShapeDtypeStruct((N, H), jnp.float32),
    )(la, lb, x, w1, b1, w2)


def _tc_scale(hw, degp):
    return pl.pallas_call(
        _scale_body,
        grid=(_GRID,),
        in_specs=[_row_spec(H), _pair_spec()],
        out_specs=_row_spec(H),
        out_shape=jax.ShapeDtypeStruct((N, H), jnp.float32),
    )(hw, degp)


def _tc_bn(agg, hwp, degp, cb, g, b, w1, b1, w2):
    return pl.pallas_call(
        _bn_body,
        grid=(2, _GRID),
        in_specs=[pl.BlockSpec((2, _BLK, H), lambda p, i: (0, i * (1 - p), 0)),
                  pl.BlockSpec((_BLK, H), lambda p, i: (i * (1 - p), 0)),
                  pl.BlockSpec((2, _BLK, H), lambda p, i: (0, i, 0)),
                  pl.BlockSpec((1, H), lambda p, i: (0, 0)),
                  pl.BlockSpec((1, H), lambda p, i: (0, 0)),
                  pl.BlockSpec((1, H), lambda p, i: (0, 0)),
                  pl.BlockSpec((H, H), lambda p, i: (0, 0)),
                  pl.BlockSpec((1, H), lambda p, i: (0, 0)),
                  pl.BlockSpec((H, H), lambda p, i: (0, 0))],
        out_specs=pl.BlockSpec((_BLK, H), lambda p, i: (i * p, 0)),
        out_shape=jax.ShapeDtypeStruct((N, H), jnp.float32),
        scratch_shapes=[pltpu.VMEM((N, H), jnp.float32),
                        pltpu.VMEM((8, H), jnp.float32)],
    )(agg, hwp, degp, cb, g, b, w1, b1, w2)


def _tc_blast(agg, hwp, degp, cb, lw, lb):
    return pl.pallas_call(
        _blast_body,
        grid=(_GRID,),
        in_specs=[_pair_spec(), _row_spec(H), _pair_spec(), _full_spec((1, H)),
                  _full_spec((H, C)), _full_spec((1, C))],
        out_specs=_row_spec(C),
        out_shape=jax.ShapeDtypeStruct((N, C), jnp.float32),
    )(agg, hwp, degp, cb, lw, lb)


def kernel(x, edge_index, params):
    ei3 = edge_index.reshape(2, _NCHUNK, _CHUNK)

    ones128 = jnp.ones((_CHUNK, H), jnp.float32)
    zeros128 = jnp.zeros((_ZCH, H), jnp.float32)

    degp = _get_sc_degree()(ei3, ones128, zeros128)

    la = params["log_alpha"].reshape(1, 1)
    lb = params["log_beta"].reshape(1, 1)

    out = None
    hwp = None
    for i in range(NUM_LAYERS):
        w1 = params["mlp_w"][i]
        b1 = params["mlp_b"][i].reshape(1, H)
        w2 = params["conv_w"][i]
        cb = params["conv_b"][i].reshape(1, H)
        if i == 0:
            hw0 = _tc_a0raw(la, lb, x, w1, b1, w2)
            hwp = _tc_scale(hw0, degp)
        agg = _get_sc_aggregate()(hwp, ei3, zeros128)
        if i < NUM_LAYERS - 1:
            hwp = _tc_bn(agg, hwp, degp, cb,
                         params["bn_gamma"][i].reshape(1, H),
                         params["bn_beta"][i].reshape(1, H),
                         params["mlp_w"][i + 1],
                         params["mlp_b"][i + 1].reshape(1, H),
                         params["conv_w"][i + 1])
        else:
            out = _tc_blast(agg, hwp, degp, cb,
                            params["lin_w"], params["lin_b"].reshape(1, C))
    return out


# speedup vs baseline: 24.3665x; 1.0095x over previous
import functools

import jax
import jax.numpy as jnp
from jax import lax
from jax.experimental import pallas as pl
from jax.experimental.pallas import tpu as pltpu
from jax.experimental.pallas import tpu_sc as plsc

N = 10000
E = 320000
D = 128
H = 128
C = 40
NUM_LAYERS = 3

_CHUNK = 128
_NCHUNK = E // _CHUNK
_NW = 32
_RPT = 624
_ZCH = 104
_TAIL = 16
_TAIL_BASE = 16 * _RPT

_BLK = 2000
_GRID = N // _BLK


def _edge_range(w):
    lo = (w * _NCHUNK) // _NW
    hi = ((w + 1) * _NCHUNK) // _NW
    return lo, hi


@functools.cache
def _get_sc_degree():
    return functools.partial(
        pl.kernel,
        out_type=jax.ShapeDtypeStruct((2, N, H), jnp.float32),
        mesh=plsc.VectorSubcoreMesh(core_axis_name="c", subcore_axis_name="s"),
        scratch_types=[
            pltpu.VMEM((_CHUNK,), jnp.int32),
            pltpu.VMEM((_CHUNK,), jnp.int32),
            pltpu.VMEM((_CHUNK,), jnp.int32),
            pltpu.VMEM((_CHUNK,), jnp.int32),
            pltpu.VMEM((_CHUNK, H), jnp.float32),
            pltpu.VMEM((_ZCH, H), jnp.float32),
            pltpu.VMEM_SHARED((N, H), jnp.float32),
            pltpu.SemaphoreType.DMA,
            pltpu.SemaphoreType.DMA,
            pltpu.SemaphoreType.DMA,
            pltpu.SemaphoreType.DMA,
        ],
    )(_sc_degree_body)


def _sc_degree_body(ei_hbm, ones_hbm, zeros_hbm, out_hbm,
                    dst0, dst1, dst2, dst3, ones_v, zero_v, acc,
                    sem_i0, sem_i1, sem_i2, sem_i3):
    c = lax.axis_index("c")
    s = lax.axis_index("s")
    dst_v = (dst0, dst1, dst2, dst3)
    sem_i = (sem_i0, sem_i1, sem_i2, sem_i3)
    pltpu.sync_copy(zeros_hbm, zero_v)
    pltpu.sync_copy(ones_hbm, ones_v)
    for j in range(_RPT // _ZCH):
        pltpu.sync_copy(zero_v, acc.at[pl.ds(s * _RPT + j * _ZCH, _ZCH)])

    @pl.when(s == 15)
    def _zero_tail():
        pltpu.sync_copy(zero_v.at[pl.ds(0, _TAIL)], acc.at[pl.ds(_TAIL_BASE, _TAIL)])

    plsc.subcore_barrier()
    w = s * 2 + c
    lo, hi = _edge_range(w)

    def load_idx(i, j):
        pltpu.async_copy(ei_hbm.at[1, i], dst_v[j], sem_i[j])

    def wait_idx(j):
        pltpu.make_async_copy(ei_hbm.at[1, 0], dst_v[j], sem_i[j]).wait()

    load_idx(lo, 0)
    load_idx(lo + 1, 1)
    load_idx(lo + 2, 2)

    def step(i, j):
        wait_idx(j)
        pltpu.sync_copy(ones_v, acc.at[dst_v[j]], add=True)

        @pl.when(i + 3 < hi)
        def _prefetch_idx():
            load_idx(i + 3, (j + 3) % 4)

    def quad(k, carry):
        i0 = lo + 4 * k
        for j in range(4):
            step(i0 + j, j)
        return carry

    nq = (hi - lo) // 4
    lax.fori_loop(0, nq, quad, 0)
    rem = (hi - lo) - 4 * nq
    for j in range(3):
        @pl.when(j < rem)
        def _tail_step(j=j):
            step(lo + 4 * nq + j, j)

    plsc.subcore_barrier()
    pltpu.sync_copy(acc.at[pl.ds(s * _RPT, _RPT)],
                    out_hbm.at[c, pl.ds(s * _RPT, _RPT)])

    @pl.when(s == 15)
    def _copy_tail():
        pltpu.sync_copy(acc.at[pl.ds(_TAIL_BASE, _TAIL)],
                        out_hbm.at[c, pl.ds(_TAIL_BASE, _TAIL)])


@functools.cache
def _get_sc_aggregate():
    return functools.partial(
        pl.kernel,
        out_type=jax.ShapeDtypeStruct((2, N, H), jnp.float32),
        mesh=plsc.VectorSubcoreMesh(core_axis_name="c", subcore_axis_name="s"),
        scratch_types=[
            pltpu.VMEM((_CHUNK,), jnp.int32),
            pltpu.VMEM((_CHUNK,), jnp.int32),
            pltpu.VMEM((_CHUNK,), jnp.int32),
            pltpu.VMEM((_CHUNK,), jnp.int32),
            pltpu.VMEM((_CHUNK,), jnp.int32),
            pltpu.VMEM((_CHUNK,), jnp.int32),
            pltpu.VMEM((_CHUNK,), jnp.int32),
            pltpu.VMEM((_CHUNK,), jnp.int32),
            pltpu.VMEM((_CHUNK, H), jnp.float32),
            pltpu.VMEM((_CHUNK, H), jnp.float32),
            pltpu.VMEM((_ZCH, H), jnp.float32),
            pltpu.VMEM_SHARED((N, H), jnp.float32),
            pltpu.SemaphoreType.DMA,
            pltpu.SemaphoreType.DMA,
            pltpu.SemaphoreType.DMA,
            pltpu.SemaphoreType.DMA,
            pltpu.SemaphoreType.DMA,
            pltpu.SemaphoreType.DMA,
        ],
    )(_sc_aggregate_body)


def _sc_aggregate_body(hwp_hbm, ei_hbm, zeros_hbm, out_hbm,
                       src0, src1, src2, src3, dst0, dst1, dst2, dst3,
                       rows0, rows1, zero_v, acc,
                       sem_i0, sem_i1, sem_i2, sem_i3, sem_g0, sem_g1):
    c = lax.axis_index("c")
    s = lax.axis_index("s")
    src_v = (src0, src1, src2, src3)
    dst_v = (dst0, dst1, dst2, dst3)
    rows_v = (rows0, rows1)
    sem_i = (sem_i0, sem_i1, sem_i2, sem_i3)
    sem_g = (sem_g0, sem_g1)

    pltpu.sync_copy(zeros_hbm, zero_v)
    for j in range(_RPT // _ZCH):
        pltpu.sync_copy(zero_v, acc.at[pl.ds(s * _RPT + j * _ZCH, _ZCH)])

    @pl.when(s == 15)
    def _zero_tail():
        pltpu.sync_copy(zero_v.at[pl.ds(0, _TAIL)], acc.at[pl.ds(_TAIL_BASE, _TAIL)])

    plsc.subcore_barrier()
    w = s * 2 + c
    lo, hi = _edge_range(w)

    def load_idx(i, j):
        pltpu.async_copy(ei_hbm.at[0, i], src_v[j], sem_i[j])
        pltpu.async_copy(ei_hbm.at[1, i], dst_v[j], sem_i[j])

    def wait_idx(j):
        pltpu.make_async_copy(ei_hbm.at[0, 0], src_v[j], sem_i[j]).wait()
        pltpu.make_async_copy(ei_hbm.at[1, 0], dst_v[j], sem_i[j]).wait()

    def start_gather(j, b):
        pltpu.async_copy(hwp_hbm.at[src_v[j]], rows_v[b], sem_g[b])

    def wait_gather(b):
        pltpu.make_async_copy(hwp_hbm.at[pl.ds(0, _CHUNK)], rows_v[b], sem_g[b]).wait()

    load_idx(lo, 0)
    load_idx(lo + 1, 1)
    load_idx(lo + 2, 2)
    wait_idx(0)
    start_gather(0, 0)

    def step(i, j):
        b = j % 2
        nb = (j + 1) % 2
        nj = (j + 1) % 4

        @pl.when(i + 1 < hi)
        def _next_gather():
            wait_idx(nj)
            start_gather(nj, nb)

        wait_gather(b)
        pltpu.sync_copy(rows_v[b], acc.at[dst_v[j]], add=True)

        @pl.when(i + 3 < hi)
        def _prefetch_idx():
            load_idx(i + 3, (j + 3) % 4)

    def quad(k, carry):
        i0 = lo + 4 * k
        for j in range(4):
            step(i0 + j, j)
        return carry

    nq = (hi - lo) // 4
    lax.fori_loop(0, nq, quad, 0)
    rem = (hi - lo) - 4 * nq
    for j in range(3):
        @pl.when(j < rem)
        def _tail_step(j=j):
            step(lo + 4 * nq + j, j)

    plsc.subcore_barrier()
    pltpu.sync_copy(acc.at[pl.ds(s * _RPT, _RPT)],
                    out_hbm.at[c, pl.ds(s * _RPT, _RPT)])

    @pl.when(s == 15)
    def _copy_tail():
        pltpu.sync_copy(acc.at[pl.ds(_TAIL_BASE, _TAIL)],
                        out_hbm.at[c, pl.ds(_TAIL_BASE, _TAIL)])


def _dinv_block(d):
    deg = 1.0 + d[0, :, 0:1] + d[1, :, 0:1]
    return lax.rsqrt(deg)


def _dinv_narrow(dv):
    return dv[:, 0:1]


def _a0raw_body(la_ref, lb_ref, x_ref, w1_ref, b1_ref, w2_ref, out_ref):
    alpha = jnp.exp(la_ref[0, 0])
    beta = jnp.exp(lb_ref[0, 0])
    h = alpha * x_ref[:, :D] + beta * x_ref[:, D:]
    m = jnp.maximum(jnp.dot(h, w1_ref[...], preferred_element_type=jnp.float32)
                    + b1_ref[...], 0.0)
    out_ref[...] = jnp.dot(m, w2_ref[...], preferred_element_type=jnp.float32)


def _scale_body(hw_ref, d_ref, out_ref, dv_ref):
    dinv = _dinv_block(d_ref[...])
    out_ref[...] = hw_ref[...] * dinv
    dv_ref[...] = jnp.broadcast_to(dinv, (_BLK, 16))


def _bn_body(agg_ref, hwp_ref, d_ref, cb_ref, g_ref, b_ref,
             w1_ref, b1_ref, w2_ref, out_ref, t_sc, stat_sc):
    p = pl.program_id(0)
    i = pl.program_id(1)
    dinv = _dinv_narrow(d_ref[...])

    @pl.when(p == 0)
    def _phase_stats():
        a = agg_ref[...]
        t = dinv * (a[0] + a[1] + hwp_ref[...]) + cb_ref[...]
        t_sc[pl.ds(i * _BLK, _BLK), :] = t
        s = jnp.sum(t, axis=0, keepdims=True)
        ss = jnp.sum(t * t, axis=0, keepdims=True)

        @pl.when(i == 0)
        def _init():
            stat_sc[0:1, :] = s
            stat_sc[1:2, :] = ss

        @pl.when(i > 0)
        def _accum():
            stat_sc[0:1, :] = stat_sc[0:1, :] + s
            stat_sc[1:2, :] = stat_sc[1:2, :] + ss

    @pl.when(p == 1)
    def _phase_apply():
        mean = stat_sc[0:1, :] * (1.0 / N)
        var = stat_sc[1:2, :] * (1.0 / N) - mean * mean
        t = t_sc[pl.ds(i * _BLK, _BLK), :]
        y = (t - mean) / jnp.sqrt(var + 1e-5) * g_ref[...] + b_ref[...]
        h = jnp.maximum(y, 0.0)
        m = jnp.maximum(jnp.dot(h, w1_ref[...], preferred_element_type=jnp.float32)
                        + b1_ref[...], 0.0)
        hw = jnp.dot(m, w2_ref[...], preferred_element_type=jnp.float32)
        out_ref[...] = hw * dinv


def _blast_body(a_ref, hwp_ref, d_ref, cb_ref, lw_ref, lb_ref, out_ref):
    a = a_ref[...]
    dinv = _dinv_narrow(d_ref[...])
    t = dinv * (a[0] + a[1] + hwp_ref[...]) + cb_ref[...]
    z = jnp.dot(t, lw_ref[...], preferred_element_type=jnp.float32) + lb_ref[...]
    z = z - jnp.max(z, axis=-1, keepdims=True)
    out_ref[...] = z - jnp.log(jnp.sum(jnp.exp(z), axis=-1, keepdims=True))


def _row_spec(width):
    return pl.BlockSpec((_BLK, width), lambda i: (i, 0))


def _pair_spec():
    return pl.BlockSpec((2, _BLK, H), lambda i: (0, i, 0))


def _deg_spec():
    return pl.BlockSpec((2, _BLK, 16), lambda i: (0, i, 0))


def _full_spec(shape):
    return pl.BlockSpec(shape, lambda i: tuple(0 for _ in shape))


_SMEM_SPEC = pl.BlockSpec(memory_space=pltpu.SMEM)


def _tc_a0raw(la, lb, x, w1, b1, w2):
    return pl.pallas_call(
        _a0raw_body,
        grid=(_GRID,),
        in_specs=[_SMEM_SPEC, _SMEM_SPEC, _row_spec(2 * D),
                  _full_spec((D, H)), _full_spec((1, H)), _full_spec((H, H))],
        out_specs=_row_spec(H),
        out_shape=jax.ShapeDtypeStruct((N, H), jnp.float32),
    )(la, lb, x, w1, b1, w2)


def _tc_scale(hw, degp):
    return pl.pallas_call(
        _scale_body,
        grid=(_GRID,),
        in_specs=[_row_spec(H), _pair_spec()],
        out_specs=[_row_spec(H), _row_spec(16)],
        out_shape=[jax.ShapeDtypeStruct((N, H), jnp.float32),
                   jax.ShapeDtypeStruct((N, 16), jnp.float32)],
    )(hw, degp)


def _tc_bn(agg, hwp, degp, cb, g, b, w1, b1, w2):
    return pl.pallas_call(
        _bn_body,
        grid=(2, _GRID),
        in_specs=[pl.BlockSpec((2, _BLK, H), lambda p, i: (0, i * (1 - p), 0)),
                  pl.BlockSpec((_BLK, H), lambda p, i: (i * (1 - p), 0)),
                  pl.BlockSpec((_BLK, 16), lambda p, i: (i, 0)),
                  pl.BlockSpec((1, H), lambda p, i: (0, 0)),
                  pl.BlockSpec((1, H), lambda p, i: (0, 0)),
                  pl.BlockSpec((1, H), lambda p, i: (0, 0)),
                  pl.BlockSpec((H, H), lambda p, i: (0, 0)),
                  pl.BlockSpec((1, H), lambda p, i: (0, 0)),
                  pl.BlockSpec((H, H), lambda p, i: (0, 0))],
        out_specs=pl.BlockSpec((_BLK, H), lambda p, i: (i * p, 0)),
        out_shape=jax.ShapeDtypeStruct((N, H), jnp.float32),
        scratch_shapes=[pltpu.VMEM((N, H), jnp.float32),
                        pltpu.VMEM((8, H), jnp.float32)],
    )(agg, hwp, degp, cb, g, b, w1, b1, w2)


def _tc_blast(agg, hwp, degp, cb, lw, lb):
    return pl.pallas_call(
        _blast_body,
        grid=(_GRID,),
        in_specs=[_pair_spec(), _row_spec(H), _row_spec(16), _full_spec((1, H)),
                  _full_spec((H, C)), _full_spec((1, C))],
        out_specs=_row_spec(C),
        out_shape=jax.ShapeDtypeStruct((N, C), jnp.float32),
    )(agg, hwp, degp, cb, lw, lb)


def kernel(x, edge_index, params):
    ei3 = edge_index.reshape(2, _NCHUNK, _CHUNK)

    ones128 = jnp.ones((_CHUNK, H), jnp.float32)
    zeros128 = jnp.zeros((_ZCH, H), jnp.float32)

    degp = _get_sc_degree()(ei3, ones128, zeros128)

    la = params["log_alpha"].reshape(1, 1)
    lb = params["log_beta"].reshape(1, 1)

    out = None
    hwp = None
    for i in range(NUM_LAYERS):
        w1 = params["mlp_w"][i]
        b1 = params["mlp_b"][i].reshape(1, H)
        w2 = params["conv_w"][i]
        cb = params["conv_b"][i].reshape(1, H)
        if i == 0:
            hw0 = _tc_a0raw(la, lb, x, w1, b1, w2)
            hwp, dinv16 = _tc_scale(hw0, degp)
        agg = _get_sc_aggregate()(hwp, ei3, zeros128)
        if i < NUM_LAYERS - 1:
            hwp = _tc_bn(agg, hwp, dinv16, cb,
                         params["bn_gamma"][i].reshape(1, H),
                         params["bn_beta"][i].reshape(1, H),
                         params["mlp_w"][i + 1],
                         params["mlp_b"][i + 1].reshape(1, H),
                         params["conv_w"][i + 1])
        else:
            out = _tc_blast(agg, hwp, dinv16, cb,
                            params["lin_w"], params["lin_b"].reshape(1, C))
    return out
